# Initial kernel scaffold; baseline (speedup 1.0000x reference)
import jax
import jax.numpy as jnp
from jax import lax
from jax.experimental import pallas as pl
from jax.experimental.pallas import tpu as pltpu
from jax.experimental.pallas import tpu_sc as plsc

N = 10000
E = 320000
D_IN = 128
D_HID = 128
D_EMB = 64

NC = 2
NS = 16
EPT = E // (NC * NS)
CH = 80
NCHUNK = EPT // CH
STRIPE = 624
TAIL = N - NS * STRIPE

_mesh = lambda: plsc.VectorSubcoreMesh(core_axis_name="c", subcore_axis_name="s")


def _scalar_scatter_body(vals_hbm, src_hbm, dst_hbm, zero_hbm, out_hbm,
                         vals_v, src_v, dst_v, msg_v, acc_sh):
    c = lax.axis_index("c")
    s = lax.axis_index("s")
    pltpu.sync_copy(vals_hbm, vals_v)
    off0 = pl.multiple_of(s * STRIPE, 8)
    pltpu.sync_copy(zero_hbm.at[pl.ds(0, STRIPE)], acc_sh.at[pl.ds(off0, STRIPE)])

    @pl.when(s == NS - 1)
    def _zero_tail():
        pltpu.sync_copy(zero_hbm.at[pl.ds(0, TAIL)], acc_sh.at[pl.ds(N - TAIL, TAIL)])

    plsc.subcore_barrier()

    ebase = pl.multiple_of((c * NS + s) * EPT, 8)

    def chunk(i, carry):
        off = pl.multiple_of(ebase + i * CH, 8)
        pltpu.sync_copy(src_hbm.at[pl.ds(off, CH)], src_v)
        pltpu.sync_copy(dst_hbm.at[pl.ds(off, CH)], dst_v)
        for g in range(CH // 16):
            idx = src_v[pl.ds(g * 16, 16)]
            msg_v[pl.ds(g * 16, 16)] = plsc.load_gather(vals_v, [idx])
        pltpu.sync_copy(msg_v, acc_sh.at[dst_v], add=True)
        return carry

    lax.fori_loop(0, NCHUNK, chunk, 0)
    plsc.subcore_barrier()

    pltpu.sync_copy(acc_sh.at[pl.ds(off0, STRIPE)], out_hbm.at[c, pl.ds(off0, STRIPE)])

    @pl.when(s == NS - 1)
    def _out_tail():
        pltpu.sync_copy(acc_sh.at[pl.ds(N - TAIL, TAIL)],
                        out_hbm.at[c, pl.ds(N - TAIL, TAIL)])


def _row_scatter_body(rows_hbm, src_hbm, dst_hbm, zero_hbm, out_hbm,
                      src_v, dst_v, rows_v, acc_sh, sem):
    c = lax.axis_index("c")
    s = lax.axis_index("s")
    off0 = pl.multiple_of(s * STRIPE, 8)
    pltpu.sync_copy(zero_hbm.at[pl.ds(0, STRIPE)], acc_sh.at[pl.ds(off0, STRIPE)])

    @pl.when(s == NS - 1)
    def _zero_tail():
        pltpu.sync_copy(zero_hbm.at[pl.ds(0, TAIL)], acc_sh.at[pl.ds(N - TAIL, TAIL)])

    plsc.subcore_barrier()

    ebase = pl.multiple_of((c * NS + s) * EPT, 8)

    def chunk(i, carry):
        off = pl.multiple_of(ebase + i * CH, 8)
        pltpu.sync_copy(src_hbm.at[pl.ds(off, CH)], src_v)
        pltpu.sync_copy(dst_hbm.at[pl.ds(off, CH)], dst_v)
        pltpu.async_copy(rows_hbm.at[src_v], rows_v, sem).wait()
        pltpu.sync_copy(rows_v, acc_sh.at[dst_v], add=True)
        return carry

    lax.fori_loop(0, NCHUNK, chunk, 0)
    plsc.subcore_barrier()

    pltpu.sync_copy(acc_sh.at[pl.ds(off0, STRIPE)], out_hbm.at[c, pl.ds(off0, STRIPE)])

    @pl.when(s == NS - 1)
    def _out_tail():
        pltpu.sync_copy(acc_sh.at[pl.ds(N - TAIL, TAIL)],
                        out_hbm.at[c, pl.ds(N - TAIL, TAIL)])


def _scalar_scatter(vals, src, dst, zero_vec):
    return pl.kernel(
        _scalar_scatter_body,
        out_type=jax.ShapeDtypeStruct((NC, N), jnp.float32),
        mesh=_mesh(),
        scratch_types=[
            pltpu.VMEM((N,), jnp.float32),
            pltpu.VMEM((CH,), jnp.int32),
            pltpu.VMEM((CH,), jnp.int32),
            pltpu.VMEM((CH,), jnp.float32),
            pltpu.VMEM_SHARED((N,), jnp.float32),
        ],
    )(vals, src, dst, zero_vec)


def _row_scatter(rows, src, dst, zero_rows):
    return pl.kernel(
        _row_scatter_body,
        out_type=jax.ShapeDtypeStruct((NC, N, D_HID), jnp.float32),
        mesh=_mesh(),
        scratch_types=[
            pltpu.VMEM((CH,), jnp.int32),
            pltpu.VMEM((CH,), jnp.int32),
            pltpu.VMEM((CH, D_HID), jnp.float32),
            pltpu.VMEM_SHARED((N, D_HID), jnp.float32),
            pltpu.SemaphoreType.DMA,
        ],
    )(rows, src, dst, zero_rows)


def _mm_body(x_ref, w_ref, o_ref):
    o_ref[...] = jnp.dot(x_ref[...], w_ref[...], preferred_element_type=jnp.float32)


def _matmul(x, w):
    return pl.pallas_call(
        _mm_body,
        out_shape=jax.ShapeDtypeStruct((x.shape[0], w.shape[1]), jnp.float32),
    )(x, w)


_RB = 400
_NG = N // _RB


def _scale_body(deg2_ref, xw_ref, w2_ref, wlin_ref, xws_ref, dinv_ref, wz_ref):
    deg = deg2_ref[0] + deg2_ref[1] + 1.0
    dinv = lax.rsqrt(deg)
    dinv_ref[...] = dinv
    xws_ref[...] = dinv * xw_ref[...]
    wz_ref[...] = jnp.dot(w2_ref[...], wlin_ref[...],
                          preferred_element_type=jnp.float32)


def _scale_stage(deg_parts, xw, W2, Wlin):
    return pl.pallas_call(
        _scale_body,
        grid=(_NG,),
        in_specs=[
            pl.BlockSpec((NC, _RB, 1), lambda i: (0, i, 0)),
            pl.BlockSpec((_RB, D_HID), lambda i: (i, 0)),
            pl.BlockSpec((D_HID, D_EMB), lambda i: (0, 0)),
            pl.BlockSpec((D_EMB, 1), lambda i: (0, 0)),
        ],
        out_specs=[
            pl.BlockSpec((_RB, D_HID), lambda i: (i, 0)),
            pl.BlockSpec((_RB, 1), lambda i: (i, 0)),
            pl.BlockSpec((D_HID, 1), lambda i: (0, 0)),
        ],
        out_shape=[
            jax.ShapeDtypeStruct((N, D_HID), jnp.float32),
            jax.ShapeDtypeStruct((N, 1), jnp.float32),
            jax.ShapeDtypeStruct((D_HID, 1), jnp.float32),
        ],
    )(deg_parts, xw, W2, Wlin)


def _mid_body(agg_ref, xws_ref, dinv_ref, b1_ref, wz_ref, zs_ref):
    dinv = dinv_ref[...]
    pre = dinv * (agg_ref[0] + agg_ref[1] + xws_ref[...]) + b1_ref[...]
    h = jnp.maximum(pre, 0.0)
    z = jnp.dot(h, wz_ref[...], preferred_element_type=jnp.float32)
    zs_ref[...] = dinv * z


def _mid_stage(agg_parts, xws, dinv, b1, wz):
    return pl.pallas_call(
        _mid_body,
        grid=(_NG,),
        in_specs=[
            pl.BlockSpec((NC, _RB, D_HID), lambda i: (0, i, 0)),
            pl.BlockSpec((_RB, D_HID), lambda i: (i, 0)),
            pl.BlockSpec((_RB, 1), lambda i: (i, 0)),
            pl.BlockSpec((1, D_HID), lambda i: (0, 0)),
            pl.BlockSpec((D_HID, 1), lambda i: (0, 0)),
        ],
        out_specs=pl.BlockSpec((_RB, 1), lambda i: (i, 0)),
        out_shape=jax.ShapeDtypeStruct((N, 1), jnp.float32),
    )(agg_parts, xws, dinv, b1, wz)


def _final_body(aggz_ref, zs_ref, dinv_ref, b2_ref, wlin_ref, blin_ref, o_ref):
    cval = jnp.dot(b2_ref[...], wlin_ref[...],
                   preferred_element_type=jnp.float32) + blin_ref[...]
    o_ref[...] = dinv_ref[...] * (aggz_ref[0] + aggz_ref[1] + zs_ref[...]) + cval


def _final_stage(aggz_parts, zs, dinv, b2, Wlin, blin):
    return pl.pallas_call(
        _final_body,
        grid=(_NG,),
        in_specs=[
            pl.BlockSpec((NC, _RB, 1), lambda i: (0, i, 0)),
            pl.BlockSpec((_RB, 1), lambda i: (i, 0)),
            pl.BlockSpec((_RB, 1), lambda i: (i, 0)),
            pl.BlockSpec((1, D_EMB), lambda i: (0, 0)),
            pl.BlockSpec((D_EMB, 1), lambda i: (0, 0)),
            pl.BlockSpec((1, 1), lambda i: (0, 0)),
        ],
        out_specs=pl.BlockSpec((_RB, 1), lambda i: (i, 0)),
        out_shape=jax.ShapeDtypeStruct((N, 1), jnp.float32),
    )(aggz_parts, zs, dinv, b2, Wlin, blin)


def kernel(x, edge_index, W1, b1, W2, b2, Wlin, blin):
    src = edge_index[0]
    dst = edge_index[1]
    ones = jnp.ones((N,), jnp.float32)
    zero_vec = jnp.zeros((STRIPE,), jnp.float32)
    zero_rows = jnp.zeros((STRIPE, D_HID), jnp.float32)

    deg_parts = _scalar_scatter(ones, src, dst, zero_vec)
    xw = _matmul(x, W1)
    xws, dinv, wz = _scale_stage(deg_parts.reshape(NC, N, 1), xw, W2, Wlin)
    agg_parts = _row_scatter(xws, src, dst, zero_rows)
    zs = _mid_stage(agg_parts, xws, dinv, b1.reshape(1, D_HID), wz)
    aggz_parts = _scalar_scatter(zs.reshape(-1), src, dst, zero_vec)
    out = _final_stage(aggz_parts.reshape(NC, N, 1), zs, dinv,
                       b2.reshape(1, D_EMB), Wlin, blin.reshape(1, 1))
    return out.reshape(-1)


# speedup vs baseline: 14.4161x; 14.4161x over previous
import jax
import jax.numpy as jnp
from jax import lax
from jax.experimental import pallas as pl
from jax.experimental.pallas import tpu as pltpu
from jax.experimental.pallas import tpu_sc as plsc

N = 10000
E = 320000
D_IN = 128
D_HID = 128
D_EMB = 64

NC = 2
NS = 16
EPT = E // (NC * NS)
CH = 80
NCHUNK = EPT // CH
STRIPE = 624
TAIL = N - NS * STRIPE
ZCH = 104

_mesh = lambda: plsc.VectorSubcoreMesh(core_axis_name="c", subcore_axis_name="s")


def _scalar_scatter_body(vals_hbm, src_hbm, dst_hbm, zero_hbm, out_hbm,
                         vals_v, src_v, dst_v, msg_v, buf_v, acc_sh):
    c = lax.axis_index("c")
    s = lax.axis_index("s")
    pltpu.sync_copy(vals_hbm, vals_v)
    off0 = pl.multiple_of(s * STRIPE, 8)
    pltpu.sync_copy(zero_hbm, buf_v)
    pltpu.sync_copy(buf_v, acc_sh.at[pl.ds(off0, STRIPE)])

    @pl.when(s == NS - 1)
    def _zero_tail():
        pltpu.sync_copy(buf_v.at[pl.ds(0, TAIL)], acc_sh.at[pl.ds(N - TAIL, TAIL)])

    plsc.subcore_barrier()

    ebase = pl.multiple_of((c * NS + s) * EPT, 8)

    def chunk(i, carry):
        off = pl.multiple_of(ebase + i * CH, 8)
        pltpu.sync_copy(src_hbm.at[pl.ds(off, CH)], src_v)
        pltpu.sync_copy(dst_hbm.at[pl.ds(off, CH)], dst_v)
        for g in range(CH // 16):
            idx = src_v[pl.ds(g * 16, 16)]
            msg_v[pl.ds(g * 16, 16)] = plsc.load_gather(vals_v, [idx])
        pltpu.sync_copy(msg_v, acc_sh.at[dst_v], add=True)
        return carry

    lax.fori_loop(0, NCHUNK, chunk, 0)
    plsc.subcore_barrier()

    obase = pl.multiple_of(c * N, 8)
    pltpu.sync_copy(acc_sh.at[pl.ds(off0, STRIPE)], buf_v)
    pltpu.sync_copy(buf_v, out_hbm.at[pl.ds(obase + off0, STRIPE)])

    @pl.when(s == NS - 1)
    def _out_tail():
        pltpu.sync_copy(acc_sh.at[pl.ds(N - TAIL, TAIL)], buf_v.at[pl.ds(0, TAIL)])
        pltpu.sync_copy(buf_v.at[pl.ds(0, TAIL)],
                        out_hbm.at[pl.ds(obase + N - TAIL, TAIL)])


def _row_scatter_body(rows_hbm, src_hbm, dst_hbm, zero_hbm, out_hbm,
                      src_v, dst_v, rows_v, buf_v, acc_sh, sem):
    c = lax.axis_index("c")
    s = lax.axis_index("s")
    off0 = pl.multiple_of(s * STRIPE, 8)
    pltpu.sync_copy(zero_hbm, buf_v)
    for t in range(STRIPE // ZCH):
        pltpu.sync_copy(buf_v, acc_sh.at[pl.ds(off0 + t * ZCH, ZCH)])

    @pl.when(s == NS - 1)
    def _zero_tail():
        pltpu.sync_copy(buf_v.at[pl.ds(0, TAIL)], acc_sh.at[pl.ds(N - TAIL, TAIL)])

    plsc.subcore_barrier()

    ebase = pl.multiple_of((c * NS + s) * EPT, 8)

    def chunk(i, carry):
        off = pl.multiple_of(ebase + i * CH, 8)
        pltpu.sync_copy(src_hbm.at[pl.ds(off, CH)], src_v)
        pltpu.sync_copy(dst_hbm.at[pl.ds(off, CH)], dst_v)
        pltpu.async_copy(rows_hbm.at[src_v], rows_v, sem).wait()
        pltpu.sync_copy(rows_v, acc_sh.at[dst_v], add=True)
        return carry

    lax.fori_loop(0, NCHUNK, chunk, 0)
    plsc.subcore_barrier()

    obase = pl.multiple_of(c * N, 8)
    for t in range(STRIPE // ZCH):
        pltpu.sync_copy(acc_sh.at[pl.ds(off0 + t * ZCH, ZCH)], buf_v)
        pltpu.sync_copy(buf_v, out_hbm.at[pl.ds(obase + off0 + t * ZCH, ZCH)])

    @pl.when(s == NS - 1)
    def _out_tail():
        pltpu.sync_copy(acc_sh.at[pl.ds(N - TAIL, TAIL)], buf_v.at[pl.ds(0, TAIL)])
        pltpu.sync_copy(buf_v.at[pl.ds(0, TAIL)],
                        out_hbm.at[pl.ds(obase + N - TAIL, TAIL)])


def _scalar_scatter(vals, src, dst, zero_vec):
    return pl.kernel(
        _scalar_scatter_body,
        out_type=jax.ShapeDtypeStruct((NC * N,), jnp.float32),
        mesh=_mesh(),
        compiler_params=pltpu.CompilerParams(needs_layout_passes=False),
        scratch_types=[
            pltpu.VMEM((N,), jnp.float32),
            pltpu.VMEM((CH,), jnp.int32),
            pltpu.VMEM((CH,), jnp.int32),
            pltpu.VMEM((CH,), jnp.float32),
            pltpu.VMEM((STRIPE,), jnp.float32),
            pltpu.VMEM_SHARED((N,), jnp.float32),
        ],
    )(vals, src, dst, zero_vec)


def _row_scatter(rows, src, dst, zero_rows):
    return pl.kernel(
        _row_scatter_body,
        out_type=jax.ShapeDtypeStruct((NC * N, D_HID), jnp.float32),
        mesh=_mesh(),
        scratch_types=[
            pltpu.VMEM((CH,), jnp.int32),
            pltpu.VMEM((CH,), jnp.int32),
            pltpu.VMEM((CH, D_HID), jnp.float32),
            pltpu.VMEM((ZCH, D_HID), jnp.float32),
            pltpu.VMEM_SHARED((N, D_HID), jnp.float32),
            pltpu.SemaphoreType.DMA,
        ],
    )(rows, src, dst, zero_rows)


def _mm_body(x_ref, w_ref, o_ref):
    o_ref[...] = jnp.dot(x_ref[...], w_ref[...], preferred_element_type=jnp.float32)


def _matmul(x, w):
    return pl.pallas_call(
        _mm_body,
        out_shape=jax.ShapeDtypeStruct((x.shape[0], w.shape[1]), jnp.float32),
    )(x, w)


_RB = 400
_NG = N // _RB


def _scale_body(deg2_ref, xw_ref, w2_ref, wlin_ref, xws_ref, dinv_ref, wz_ref):
    deg = deg2_ref[0] + deg2_ref[1] + 1.0
    dinv = lax.rsqrt(deg)
    dinv_ref[...] = dinv
    xws_ref[...] = dinv * xw_ref[...]
    wz_ref[...] = jnp.dot(w2_ref[...], wlin_ref[...],
                          preferred_element_type=jnp.float32)


def _scale_stage(deg_parts, xw, W2, Wlin):
    return pl.pallas_call(
        _scale_body,
        grid=(_NG,),
        in_specs=[
            pl.BlockSpec((NC, _RB, 1), lambda i: (0, i, 0)),
            pl.BlockSpec((_RB, D_HID), lambda i: (i, 0)),
            pl.BlockSpec((D_HID, D_EMB), lambda i: (0, 0)),
            pl.BlockSpec((D_EMB, 1), lambda i: (0, 0)),
        ],
        out_specs=[
            pl.BlockSpec((_RB, D_HID), lambda i: (i, 0)),
            pl.BlockSpec((_RB, 1), lambda i: (i, 0)),
            pl.BlockSpec((D_HID, 1), lambda i: (0, 0)),
        ],
        out_shape=[
            jax.ShapeDtypeStruct((N, D_HID), jnp.float32),
            jax.ShapeDtypeStruct((N, 1), jnp.float32),
            jax.ShapeDtypeStruct((D_HID, 1), jnp.float32),
        ],
    )(deg_parts, xw, W2, Wlin)


def _mid_body(agg_ref, xws_ref, dinv_ref, b1_ref, wz_ref, zs_ref):
    dinv = dinv_ref[...]
    pre = dinv * (agg_ref[0] + agg_ref[1] + xws_ref[...]) + b1_ref[...]
    h = jnp.maximum(pre, 0.0)
    z = jnp.dot(h, wz_ref[...], preferred_element_type=jnp.float32)
    zs_ref[...] = dinv * z


def _mid_stage(agg_parts, xws, dinv, b1, wz):
    return pl.pallas_call(
        _mid_body,
        grid=(_NG,),
        in_specs=[
            pl.BlockSpec((NC, _RB, D_HID), lambda i: (0, i, 0)),
            pl.BlockSpec((_RB, D_HID), lambda i: (i, 0)),
            pl.BlockSpec((_RB, 1), lambda i: (i, 0)),
            pl.BlockSpec((1, D_HID), lambda i: (0, 0)),
            pl.BlockSpec((D_HID, 1), lambda i: (0, 0)),
        ],
        out_specs=pl.BlockSpec((_RB, 1), lambda i: (i, 0)),
        out_shape=jax.ShapeDtypeStruct((N, 1), jnp.float32),
    )(agg_parts, xws, dinv, b1, wz)


def _final_body(aggz_ref, zs_ref, dinv_ref, b2_ref, wlin_ref, blin_ref, o_ref):
    cval = jnp.dot(b2_ref[...], wlin_ref[...],
                   preferred_element_type=jnp.float32) + blin_ref[...]
    o_ref[...] = dinv_ref[...] * (aggz_ref[0] + aggz_ref[1] + zs_ref[...]) + cval


def _final_stage(aggz_parts, zs, dinv, b2, Wlin, blin):
    return pl.pallas_call(
        _final_body,
        grid=(_NG,),
        in_specs=[
            pl.BlockSpec((NC, _RB, 1), lambda i: (0, i, 0)),
            pl.BlockSpec((_RB, 1), lambda i: (i, 0)),
            pl.BlockSpec((_RB, 1), lambda i: (i, 0)),
            pl.BlockSpec((1, D_EMB), lambda i: (0, 0)),
            pl.BlockSpec((D_EMB, 1), lambda i: (0, 0)),
            pl.BlockSpec((1, 1), lambda i: (0, 0)),
        ],
        out_specs=pl.BlockSpec((_RB, 1), lambda i: (i, 0)),
        out_shape=jax.ShapeDtypeStruct((N, 1), jnp.float32),
    )(aggz_parts, zs, dinv, b2, Wlin, blin)


def kernel(x, edge_index, W1, b1, W2, b2, Wlin, blin):
    src = edge_index[0]
    dst = edge_index[1]
    ones = jnp.ones((N,), jnp.float32)
    zero_vec = jnp.zeros((STRIPE,), jnp.float32)
    zero_rows = jnp.zeros((ZCH, D_HID), jnp.float32)

    deg_parts = _scalar_scatter(ones, src, dst, zero_vec)
    xw = _matmul(x, W1)
    xws, dinv, wz = _scale_stage(deg_parts.reshape(NC, N, 1), xw, W2, Wlin)
    agg_parts = _row_scatter(xws, src, dst, zero_rows)
    zs = _mid_stage(agg_parts.reshape(NC, N, D_HID), xws, dinv,
                    b1.reshape(1, D_HID), wz)
    aggz_parts = _scalar_scatter(zs.reshape(-1), src, dst, zero_vec)
    out = _final_stage(aggz_parts.reshape(NC, N, 1), zs, dinv,
                       b2.reshape(1, D_EMB), Wlin, blin.reshape(1, 1))
    return out.reshape(-1)


# speedup vs baseline: 34.0618x; 2.3628x over previous
import jax
import jax.numpy as jnp
from jax import lax
from jax.experimental import pallas as pl
from jax.experimental.pallas import tpu as pltpu
from jax.experimental.pallas import tpu_sc as plsc

N = 10000
E = 320000
D_IN = 128
D_HID = 128
D_EMB = 64

NC = 2
NS = 16
EPT = E // (NC * NS)
CH = 80
NCHUNK = EPT // CH
STRIPE = 624
TAIL = N - NS * STRIPE

_mesh = lambda: plsc.VectorSubcoreMesh(core_axis_name="c", subcore_axis_name="s")
_params = lambda: pltpu.CompilerParams(needs_layout_passes=False)


def _stage_indices(src_hbm, dst_hbm, src_all, dst_all, c, s):
    ebase = pl.multiple_of((c * NS + s) * EPT, 8)
    pltpu.sync_copy(src_hbm.at[pl.ds(ebase, EPT)], src_all)
    pltpu.sync_copy(dst_hbm.at[pl.ds(ebase, EPT)], dst_all)


def _zero_acc_1d(zero_hbm, buf_v, acc_sh, s):
    off0 = pl.multiple_of(s * STRIPE, 8)
    pltpu.sync_copy(zero_hbm, buf_v)
    pltpu.sync_copy(buf_v, acc_sh.at[pl.ds(off0, STRIPE)])

    @pl.when(s == NS - 1)
    def _zero_tail():
        pltpu.sync_copy(buf_v.at[pl.ds(0, TAIL)], acc_sh.at[pl.ds(N - TAIL, TAIL)])


def _readback_1d(acc_sh, buf_v, out_hbm, c, s):
    off0 = pl.multiple_of(s * STRIPE, 8)
    obase = pl.multiple_of(c * N, 8)
    pltpu.sync_copy(acc_sh.at[pl.ds(off0, STRIPE)], buf_v)
    pltpu.sync_copy(buf_v, out_hbm.at[pl.ds(obase + off0, STRIPE)])

    @pl.when(s == NS - 1)
    def _out_tail():
        pltpu.sync_copy(acc_sh.at[pl.ds(N - TAIL, TAIL)], buf_v.at[pl.ds(0, TAIL)])
        pltpu.sync_copy(buf_v.at[pl.ds(0, TAIL)],
                        out_hbm.at[pl.ds(obase + N - TAIL, TAIL)])


def _deg_body(src_hbm, dst_hbm, zero_hbm, out_hbm,
              src_all, dst_all, dstb, msg_v, buf_v, acc_sh):
    c = lax.axis_index("c")
    s = lax.axis_index("s")
    _stage_indices(src_hbm, dst_hbm, src_all, dst_all, c, s)
    _zero_acc_1d(zero_hbm, buf_v, acc_sh, s)
    for k in range(CH // 16):
        msg_v[pl.ds(k * 16, 16)] = jnp.ones((16,), jnp.float32)
    plsc.subcore_barrier()

    def chunk(i, carry):
        off = pl.multiple_of(i * CH, 8)
        for k in range(CH // 16):
            dstb[pl.ds(k * 16, 16)] = dst_all[pl.ds(off + k * 16, 16)]
        pltpu.sync_copy(msg_v, acc_sh.at[dstb], add=True)
        return carry

    lax.fori_loop(0, NCHUNK, chunk, 0)
    plsc.subcore_barrier()
    _readback_1d(acc_sh, buf_v, out_hbm, c, s)


def _deg_counts(src, dst, zero_vec):
    return pl.kernel(
        _deg_body,
        out_type=jax.ShapeDtypeStruct((NC * N,), jnp.float32),
        mesh=_mesh(),
        compiler_params=_params(),
        scratch_types=[
            pltpu.VMEM((EPT,), jnp.int32),
            pltpu.VMEM((EPT,), jnp.int32),
            pltpu.VMEM((CH,), jnp.int32),
            pltpu.VMEM((CH,), jnp.float32),
            pltpu.VMEM((STRIPE,), jnp.float32),
            pltpu.VMEM_SHARED((N,), jnp.float32),
        ],
    )(src, dst, zero_vec)


def _scalar_scatter_body(vals_hbm, src_hbm, dst_hbm, zero_hbm, out_hbm,
                         vals_v, src_all, dst_all, dstb, msg_v, buf_v, acc_sh):
    c = lax.axis_index("c")
    s = lax.axis_index("s")
    pltpu.sync_copy(vals_hbm, vals_v)
    _stage_indices(src_hbm, dst_hbm, src_all, dst_all, c, s)
    _zero_acc_1d(zero_hbm, buf_v, acc_sh, s)
    plsc.subcore_barrier()

    def chunk(i, carry):
        off = pl.multiple_of(i * CH, 8)
        for k in range(CH // 16):
            idx = src_all[pl.ds(off + k * 16, 16)]
            msg_v[pl.ds(k * 16, 16)] = plsc.load_gather(vals_v, [idx])
            dstb[pl.ds(k * 16, 16)] = dst_all[pl.ds(off + k * 16, 16)]
        pltpu.sync_copy(msg_v, acc_sh.at[dstb], add=True)
        return carry

    lax.fori_loop(0, NCHUNK, chunk, 0)
    plsc.subcore_barrier()
    _readback_1d(acc_sh, buf_v, out_hbm, c, s)


def _scalar_scatter(vals, src, dst, zero_vec):
    return pl.kernel(
        _scalar_scatter_body,
        out_type=jax.ShapeDtypeStruct((NC * N,), jnp.float32),
        mesh=_mesh(),
        compiler_params=_params(),
        scratch_types=[
            pltpu.VMEM((N,), jnp.float32),
            pltpu.VMEM((EPT,), jnp.int32),
            pltpu.VMEM((EPT,), jnp.int32),
            pltpu.VMEM((CH,), jnp.int32),
            pltpu.VMEM((CH,), jnp.float32),
            pltpu.VMEM((STRIPE,), jnp.float32),
            pltpu.VMEM_SHARED((N,), jnp.float32),
        ],
    )(vals, src, dst, zero_vec)


def _row_scatter_body(rows_hbm, src_hbm, dst_hbm, zero_hbm, out_hbm,
                      src_all, dst_all, dstb0, dstb1, rows0, rows1,
                      acc_sh, sem0, sem1):
    c = lax.axis_index("c")
    s = lax.axis_index("s")
    _stage_indices(src_hbm, dst_hbm, src_all, dst_all, c, s)
    off0 = pl.multiple_of(s * STRIPE, 8)
    pltpu.sync_copy(zero_hbm, rows0)
    for t in range(STRIPE // CH):
        pltpu.sync_copy(rows0, acc_sh.at[pl.ds(off0 + t * CH, CH)])
    rem = STRIPE - (STRIPE // CH) * CH
    pltpu.sync_copy(rows0.at[pl.ds(0, rem)],
                    acc_sh.at[pl.ds(off0 + STRIPE - rem, rem)])

    @pl.when(s == NS - 1)
    def _zero_tail():
        pltpu.sync_copy(rows0.at[pl.ds(0, TAIL)], acc_sh.at[pl.ds(N - TAIL, TAIL)])

    plsc.subcore_barrier()

    def fill_dstb(dstb, off):
        for k in range(CH // 16):
            dstb[pl.ds(k * 16, 16)] = dst_all[pl.ds(off + k * 16, 16)]

    def gather(rows_v, off, sem):
        return pltpu.async_copy(rows_hbm.at[src_all.at[pl.ds(off, CH)]],
                                rows_v, sem)

    fill_dstb(dstb0, 0)
    gather(rows0, 0, sem0)

    def pair(g, carry):
        o0 = pl.multiple_of(2 * g * CH, 8)
        o1 = pl.multiple_of((2 * g + 1) * CH, 8)
        o2 = pl.multiple_of((2 * g + 2) * CH, 8)
        fill_dstb(dstb1, o1)
        cp1 = gather(rows1, o1, sem1)
        pltpu.make_async_copy(rows_hbm.at[src_all.at[pl.ds(o0, CH)]],
                              rows0, sem0).wait()
        pltpu.sync_copy(rows0, acc_sh.at[dstb0], add=True)
        fill_dstb(dstb0, o2)
        cp2 = gather(rows0, o2, sem0)
        cp1.wait()
        pltpu.sync_copy(rows1, acc_sh.at[dstb1], add=True)
        return carry

    lax.fori_loop(0, (NCHUNK - 1) // 2, pair, 0)
    olast = pl.multiple_of((NCHUNK - 1) * CH, 8)
    pltpu.make_async_copy(rows_hbm.at[src_all.at[pl.ds(olast, CH)]],
                          rows0, sem0).wait()
    pltpu.sync_copy(rows0, acc_sh.at[dstb0], add=True)

    plsc.subcore_barrier()
    obase = pl.multiple_of(c * N, 8)
    for t in range(STRIPE // CH):
        pltpu.sync_copy(acc_sh.at[pl.ds(off0 + t * CH, CH)], rows0)
        pltpu.sync_copy(rows0, out_hbm.at[pl.ds(obase + off0 + t * CH, CH)])
    pltpu.sync_copy(acc_sh.at[pl.ds(off0 + STRIPE - rem, rem)],
                    rows0.at[pl.ds(0, rem)])
    pltpu.sync_copy(rows0.at[pl.ds(0, rem)],
                    out_hbm.at[pl.ds(obase + off0 + STRIPE - rem, rem)])

    @pl.when(s == NS - 1)
    def _out_tail():
        pltpu.sync_copy(acc_sh.at[pl.ds(N - TAIL, TAIL)], rows1.at[pl.ds(0, TAIL)])
        pltpu.sync_copy(rows1.at[pl.ds(0, TAIL)],
                        out_hbm.at[pl.ds(obase + N - TAIL, TAIL)])


def _row_scatter(rows, src, dst, zero_rows):
    return pl.kernel(
        _row_scatter_body,
        out_type=jax.ShapeDtypeStruct((NC * N, D_HID), jnp.float32),
        mesh=_mesh(),
        compiler_params=_params(),
        scratch_types=[
            pltpu.VMEM((EPT,), jnp.int32),
            pltpu.VMEM((EPT,), jnp.int32),
            pltpu.VMEM((CH,), jnp.int32),
            pltpu.VMEM((CH,), jnp.int32),
            pltpu.VMEM((CH, D_HID), jnp.float32),
            pltpu.VMEM((CH, D_HID), jnp.float32),
            pltpu.VMEM_SHARED((N, D_HID), jnp.float32),
            pltpu.SemaphoreType.DMA,
            pltpu.SemaphoreType.DMA,
        ],
    )(rows, src, dst, zero_rows)


def _mm_body(x_ref, w_ref, o_ref):
    o_ref[...] = jnp.dot(x_ref[...], w_ref[...], preferred_element_type=jnp.float32)


def _matmul(x, w):
    return pl.pallas_call(
        _mm_body,
        out_shape=jax.ShapeDtypeStruct((x.shape[0], w.shape[1]), jnp.float32),
    )(x, w)


_RB = 400
_NG = N // _RB


def _scale_body(deg2_ref, xw_ref, w2_ref, wlin_ref, xws_ref, dinv_ref, wz_ref):
    deg = deg2_ref[0] + deg2_ref[1] + 1.0
    dinv = lax.rsqrt(deg)
    dinv_ref[...] = dinv
    xws_ref[...] = dinv * xw_ref[...]
    wz_ref[...] = jnp.dot(w2_ref[...], wlin_ref[...],
                          preferred_element_type=jnp.float32)


def _scale_stage(deg_parts, xw, W2, Wlin):
    return pl.pallas_call(
        _scale_body,
        grid=(_NG,),
        in_specs=[
            pl.BlockSpec((NC, _RB, 1), lambda i: (0, i, 0)),
            pl.BlockSpec((_RB, D_HID), lambda i: (i, 0)),
            pl.BlockSpec((D_HID, D_EMB), lambda i: (0, 0)),
            pl.BlockSpec((D_EMB, 1), lambda i: (0, 0)),
        ],
        out_specs=[
            pl.BlockSpec((_RB, D_HID), lambda i: (i, 0)),
            pl.BlockSpec((_RB, 1), lambda i: (i, 0)),
            pl.BlockSpec((D_HID, 1), lambda i: (0, 0)),
        ],
        out_shape=[
            jax.ShapeDtypeStruct((N, D_HID), jnp.float32),
            jax.ShapeDtypeStruct((N, 1), jnp.float32),
            jax.ShapeDtypeStruct((D_HID, 1), jnp.float32),
        ],
    )(deg_parts, xw, W2, Wlin)


def _mid_body(agg_ref, xws_ref, dinv_ref, b1_ref, wz_ref, zs_ref):
    dinv = dinv_ref[...]
    pre = dinv * (agg_ref[0] + agg_ref[1] + xws_ref[...]) + b1_ref[...]
    h = jnp.maximum(pre, 0.0)
    z = jnp.dot(h, wz_ref[...], preferred_element_type=jnp.float32)
    zs_ref[...] = dinv * z


def _mid_stage(agg_parts, xws, dinv, b1, wz):
    return pl.pallas_call(
        _mid_body,
        grid=(_NG,),
        in_specs=[
            pl.BlockSpec((NC, _RB, D_HID), lambda i: (0, i, 0)),
            pl.BlockSpec((_RB, D_HID), lambda i: (i, 0)),
            pl.BlockSpec((_RB, 1), lambda i: (i, 0)),
            pl.BlockSpec((1, D_HID), lambda i: (0, 0)),
            pl.BlockSpec((D_HID, 1), lambda i: (0, 0)),
        ],
        out_specs=pl.BlockSpec((_RB, 1), lambda i: (i, 0)),
        out_shape=jax.ShapeDtypeStruct((N, 1), jnp.float32),
    )(agg_parts, xws, dinv, b1, wz)


def _final_body(aggz_ref, zs_ref, dinv_ref, b2_ref, wlin_ref, blin_ref, o_ref):
    cval = jnp.dot(b2_ref[...], wlin_ref[...],
                   preferred_element_type=jnp.float32) + blin_ref[...]
    o_ref[...] = dinv_ref[...] * (aggz_ref[0] + aggz_ref[1] + zs_ref[...]) + cval


def _final_stage(aggz_parts, zs, dinv, b2, Wlin, blin):
    return pl.pallas_call(
        _final_body,
        grid=(_NG,),
        in_specs=[
            pl.BlockSpec((NC, _RB, 1), lambda i: (0, i, 0)),
            pl.BlockSpec((_RB, 1), lambda i: (i, 0)),
            pl.BlockSpec((_RB, 1), lambda i: (i, 0)),
            pl.BlockSpec((1, D_EMB), lambda i: (0, 0)),
            pl.BlockSpec((D_EMB, 1), lambda i: (0, 0)),
            pl.BlockSpec((1, 1), lambda i: (0, 0)),
        ],
        out_specs=pl.BlockSpec((_RB, 1), lambda i: (i, 0)),
        out_shape=jax.ShapeDtypeStruct((N, 1), jnp.float32),
    )(aggz_parts, zs, dinv, b2, Wlin, blin)


def kernel(x, edge_index, W1, b1, W2, b2, Wlin, blin):
    src = edge_index[0]
    dst = edge_index[1]
    zero_vec = jnp.zeros((STRIPE,), jnp.float32)
    zero_rows = jnp.zeros((CH, D_HID), jnp.float32)

    deg_parts = _deg_counts(src, dst, zero_vec)
    xw = _matmul(x, W1)
    xws, dinv, wz = _scale_stage(deg_parts.reshape(NC, N, 1), xw, W2, Wlin)
    agg_parts = _row_scatter(xws, src, dst, zero_rows)
    zs = _mid_stage(agg_parts.reshape(NC, N, D_HID), xws, dinv,
                    b1.reshape(1, D_HID), wz)
    aggz_parts = _scalar_scatter(zs.reshape(-1), src, dst, zero_vec)
    out = _final_stage(aggz_parts.reshape(NC, N, 1), zs, dinv,
                       b2.reshape(1, D_EMB), Wlin, blin.reshape(1, 1))
    return out.reshape(-1)


# speedup vs baseline: 35.6753x; 1.0474x over previous
import jax
import jax.numpy as jnp
from jax import lax
from jax.experimental import pallas as pl
from jax.experimental.pallas import tpu as pltpu
from jax.experimental.pallas import tpu_sc as plsc

N = 10000
E = 320000
D_IN = 128
D_HID = 128
D_EMB = 64

NC = 2
NS = 16
EPT = E // (NC * NS)
CH = 80
NCHUNK = EPT // CH
STRIPE = 624
TAIL = N - NS * STRIPE

_mesh = lambda: plsc.VectorSubcoreMesh(core_axis_name="c", subcore_axis_name="s")
_params = lambda: pltpu.CompilerParams(needs_layout_passes=False)


def _stage_indices(src_hbm, dst_hbm, src_all, dst_all, c, s):
    ebase = pl.multiple_of((c * NS + s) * EPT, 8)
    pltpu.sync_copy(src_hbm.at[pl.ds(ebase, EPT)], src_all)
    pltpu.sync_copy(dst_hbm.at[pl.ds(ebase, EPT)], dst_all)


def _zero_acc_1d(zero_hbm, buf_v, acc_sh, s):
    off0 = pl.multiple_of(s * STRIPE, 8)
    pltpu.sync_copy(zero_hbm, buf_v)
    pltpu.sync_copy(buf_v, acc_sh.at[pl.ds(off0, STRIPE)])

    @pl.when(s == NS - 1)
    def _zero_tail():
        pltpu.sync_copy(buf_v.at[pl.ds(0, TAIL)], acc_sh.at[pl.ds(N - TAIL, TAIL)])


def _readback_1d(acc_sh, buf_v, out_hbm, c, s):
    off0 = pl.multiple_of(s * STRIPE, 8)
    obase = pl.multiple_of(c * N, 8)
    pltpu.sync_copy(acc_sh.at[pl.ds(off0, STRIPE)], buf_v)
    pltpu.sync_copy(buf_v, out_hbm.at[pl.ds(obase + off0, STRIPE)])

    @pl.when(s == NS - 1)
    def _out_tail():
        pltpu.sync_copy(acc_sh.at[pl.ds(N - TAIL, TAIL)], buf_v.at[pl.ds(0, TAIL)])
        pltpu.sync_copy(buf_v.at[pl.ds(0, TAIL)],
                        out_hbm.at[pl.ds(obase + N - TAIL, TAIL)])


def _deg_body(src_hbm, dst_hbm, zero_hbm, out_hbm,
              src_all, dst_all, dstb, msg_v, buf_v, acc_sh):
    c = lax.axis_index("c")
    s = lax.axis_index("s")
    _stage_indices(src_hbm, dst_hbm, src_all, dst_all, c, s)
    _zero_acc_1d(zero_hbm, buf_v, acc_sh, s)
    for k in range(CH // 16):
        msg_v[pl.ds(k * 16, 16)] = jnp.ones((16,), jnp.float32)
    plsc.subcore_barrier()

    def chunk(i, carry):
        off = pl.multiple_of(i * CH, 8)
        for k in range(CH // 16):
            dstb[pl.ds(k * 16, 16)] = dst_all[pl.ds(off + k * 16, 16)]
        pltpu.sync_copy(msg_v, acc_sh.at[dstb], add=True)
        return carry

    lax.fori_loop(0, NCHUNK, chunk, 0)
    plsc.subcore_barrier()
    _readback_1d(acc_sh, buf_v, out_hbm, c, s)


def _deg_counts(src, dst, zero_vec):
    return pl.kernel(
        _deg_body,
        out_type=jax.ShapeDtypeStruct((NC * N,), jnp.float32),
        mesh=_mesh(),
        compiler_params=_params(),
        scratch_types=[
            pltpu.VMEM((EPT,), jnp.int32),
            pltpu.VMEM((EPT,), jnp.int32),
            pltpu.VMEM((CH,), jnp.int32),
            pltpu.VMEM((CH,), jnp.float32),
            pltpu.VMEM((STRIPE,), jnp.float32),
            pltpu.VMEM_SHARED((N,), jnp.float32),
        ],
    )(src, dst, zero_vec)


def _scalar_scatter_body(vals_hbm, src_hbm, dst_hbm, zero_hbm, out_hbm,
                         vals_v, src_all, dst_all, dstb, msg_v, buf_v, acc_sh):
    c = lax.axis_index("c")
    s = lax.axis_index("s")
    pltpu.sync_copy(vals_hbm, vals_v)
    _stage_indices(src_hbm, dst_hbm, src_all, dst_all, c, s)
    _zero_acc_1d(zero_hbm, buf_v, acc_sh, s)
    plsc.subcore_barrier()

    def chunk(i, carry):
        off = pl.multiple_of(i * CH, 8)
        for k in range(CH // 16):
            idx = src_all[pl.ds(off + k * 16, 16)]
            msg_v[pl.ds(k * 16, 16)] = plsc.load_gather(vals_v, [idx])
            dstb[pl.ds(k * 16, 16)] = dst_all[pl.ds(off + k * 16, 16)]
        pltpu.sync_copy(msg_v, acc_sh.at[dstb], add=True)
        return carry

    lax.fori_loop(0, NCHUNK, chunk, 0)
    plsc.subcore_barrier()
    _readback_1d(acc_sh, buf_v, out_hbm, c, s)


def _scalar_scatter(vals, src, dst, zero_vec):
    return pl.kernel(
        _scalar_scatter_body,
        out_type=jax.ShapeDtypeStruct((NC * N,), jnp.float32),
        mesh=_mesh(),
        compiler_params=_params(),
        scratch_types=[
            pltpu.VMEM((N,), jnp.float32),
            pltpu.VMEM((EPT,), jnp.int32),
            pltpu.VMEM((EPT,), jnp.int32),
            pltpu.VMEM((CH,), jnp.int32),
            pltpu.VMEM((CH,), jnp.float32),
            pltpu.VMEM((STRIPE,), jnp.float32),
            pltpu.VMEM_SHARED((N,), jnp.float32),
        ],
    )(vals, src, dst, zero_vec)


RCH = 128
RNCH = EPT // RCH
RTAIL = EPT - RNCH * RCH


def _row_scatter_body(rows_hbm, src_hbm, dst_hbm, zero_hbm, out_hbm,
                      src_all, dstb0, dstb1, dstbt, rows0, rows1,
                      acc_sh, sem0, sem1, semd0, semd1):
    c = lax.axis_index("c")
    s = lax.axis_index("s")
    ebase = pl.multiple_of((c * NS + s) * EPT, 8)
    pltpu.sync_copy(src_hbm.at[pl.ds(ebase, EPT)], src_all)
    off0 = pl.multiple_of(s * STRIPE, 8)
    pltpu.sync_copy(zero_hbm, rows0)
    for t in range(STRIPE // RCH):
        pltpu.sync_copy(rows0, acc_sh.at[pl.ds(off0 + t * RCH, RCH)])
    rem = STRIPE - (STRIPE // RCH) * RCH
    pltpu.sync_copy(rows0.at[pl.ds(0, rem)],
                    acc_sh.at[pl.ds(off0 + STRIPE - rem, rem)])

    @pl.when(s == NS - 1)
    def _zero_tail():
        pltpu.sync_copy(rows0.at[pl.ds(0, TAIL)], acc_sh.at[pl.ds(N - TAIL, TAIL)])

    plsc.subcore_barrier()

    def issue(rows_v, dstb, off, sem, semd):
        pltpu.async_copy(dst_hbm.at[pl.ds(ebase + off, RCH)], dstb, semd)
        pltpu.async_copy(rows_hbm.at[src_all.at[pl.ds(off, RCH)]], rows_v, sem)

    def drain(rows_v, dstb, off, sem, semd):
        pltpu.make_async_copy(dst_hbm.at[pl.ds(ebase + off, RCH)], dstb,
                              semd).wait()
        pltpu.make_async_copy(rows_hbm.at[src_all.at[pl.ds(off, RCH)]],
                              rows_v, sem).wait()
        pltpu.sync_copy(rows_v, acc_sh.at[dstb], add=True)

    issue(rows0, dstb0, 0, sem0, semd0)

    def pair(g, carry):
        o0 = pl.multiple_of(2 * g * RCH, 8)
        o1 = pl.multiple_of((2 * g + 1) * RCH, 8)
        o2 = pl.multiple_of((2 * g + 2) * RCH, 8)
        issue(rows1, dstb1, o1, sem1, semd1)
        drain(rows0, dstb0, o0, sem0, semd0)
        issue(rows0, dstb0, o2, sem0, semd0)
        drain(rows1, dstb1, o1, sem1, semd1)
        return carry

    lax.fori_loop(0, RNCH // 2 - 1, pair, 0)
    o76 = pl.multiple_of((RNCH - 2) * RCH, 8)
    o77 = pl.multiple_of((RNCH - 1) * RCH, 8)
    issue(rows1, dstb1, o77, sem1, semd1)
    drain(rows0, dstb0, o76, sem0, semd0)
    drain(rows1, dstb1, o77, sem1, semd1)
    ot = pl.multiple_of(RNCH * RCH, 8)
    pltpu.sync_copy(dst_hbm.at[pl.ds(ebase + ot, RTAIL)], dstbt)
    pltpu.async_copy(rows_hbm.at[src_all.at[pl.ds(ot, RTAIL)]],
                     rows0.at[pl.ds(0, RTAIL)], sem0).wait()
    pltpu.sync_copy(rows0.at[pl.ds(0, RTAIL)], acc_sh.at[dstbt], add=True)

    plsc.subcore_barrier()
    obase = pl.multiple_of(c * N, 8)
    for t in range(STRIPE // RCH):
        pltpu.sync_copy(acc_sh.at[pl.ds(off0 + t * RCH, RCH)], rows0)
        pltpu.sync_copy(rows0, out_hbm.at[pl.ds(obase + off0 + t * RCH, RCH)])
    pltpu.sync_copy(acc_sh.at[pl.ds(off0 + STRIPE - rem, rem)],
                    rows0.at[pl.ds(0, rem)])
    pltpu.sync_copy(rows0.at[pl.ds(0, rem)],
                    out_hbm.at[pl.ds(obase + off0 + STRIPE - rem, rem)])

    @pl.when(s == NS - 1)
    def _out_tail():
        pltpu.sync_copy(acc_sh.at[pl.ds(N - TAIL, TAIL)], rows1.at[pl.ds(0, TAIL)])
        pltpu.sync_copy(rows1.at[pl.ds(0, TAIL)],
                        out_hbm.at[pl.ds(obase + N - TAIL, TAIL)])


def _row_scatter(rows, src, dst, zero_rows):
    return pl.kernel(
        _row_scatter_body,
        out_type=jax.ShapeDtypeStruct((NC * N, D_HID), jnp.float32),
        mesh=_mesh(),
        compiler_params=_params(),
        scratch_types=[
            pltpu.VMEM((EPT,), jnp.int32),
            pltpu.VMEM((RCH,), jnp.int32),
            pltpu.VMEM((RCH,), jnp.int32),
            pltpu.VMEM((RTAIL,), jnp.int32),
            pltpu.VMEM((RCH, D_HID), jnp.float32),
            pltpu.VMEM((RCH, D_HID), jnp.float32),
            pltpu.VMEM_SHARED((N, D_HID), jnp.float32),
            pltpu.SemaphoreType.DMA,
            pltpu.SemaphoreType.DMA,
            pltpu.SemaphoreType.DMA,
            pltpu.SemaphoreType.DMA,
        ],
    )(rows, src, dst, zero_rows)


_RB = 400
_NG = N // _RB


def _scale_body(deg2_ref, x_ref, w1_ref, w2_ref, wlin_ref,
                xws_ref, dinv_ref, wz_ref):
    deg = deg2_ref[0] + deg2_ref[1] + 1.0
    dinv = lax.rsqrt(deg)
    dinv_ref[...] = dinv
    xw = jnp.dot(x_ref[...], w1_ref[...], preferred_element_type=jnp.float32)
    xws_ref[...] = dinv * xw
    wz_ref[...] = jnp.dot(w2_ref[...], wlin_ref[...],
                          preferred_element_type=jnp.float32)


def _scale_stage(deg_parts, x, W1, W2, Wlin):
    return pl.pallas_call(
        _scale_body,
        grid=(_NG,),
        in_specs=[
            pl.BlockSpec((NC, _RB, 1), lambda i: (0, i, 0)),
            pl.BlockSpec((_RB, D_IN), lambda i: (i, 0)),
            pl.BlockSpec((D_IN, D_HID), lambda i: (0, 0)),
            pl.BlockSpec((D_HID, D_EMB), lambda i: (0, 0)),
            pl.BlockSpec((D_EMB, 1), lambda i: (0, 0)),
        ],
        out_specs=[
            pl.BlockSpec((_RB, D_HID), lambda i: (i, 0)),
            pl.BlockSpec((_RB, 1), lambda i: (i, 0)),
            pl.BlockSpec((D_HID, 1), lambda i: (0, 0)),
        ],
        out_shape=[
            jax.ShapeDtypeStruct((N, D_HID), jnp.float32),
            jax.ShapeDtypeStruct((N, 1), jnp.float32),
            jax.ShapeDtypeStruct((D_HID, 1), jnp.float32),
        ],
    )(deg_parts, x, W1, W2, Wlin)


def _mid_body(agg_ref, xws_ref, dinv_ref, b1_ref, wz_ref, zs_ref):
    dinv = dinv_ref[...]
    pre = dinv * (agg_ref[0] + agg_ref[1] + xws_ref[...]) + b1_ref[...]
    h = jnp.maximum(pre, 0.0)
    z = jnp.dot(h, wz_ref[...], preferred_element_type=jnp.float32)
    zs_ref[...] = dinv * z


def _mid_stage(agg_parts, xws, dinv, b1, wz):
    return pl.pallas_call(
        _mid_body,
        grid=(_NG,),
        in_specs=[
            pl.BlockSpec((NC, _RB, D_HID), lambda i: (0, i, 0)),
            pl.BlockSpec((_RB, D_HID), lambda i: (i, 0)),
            pl.BlockSpec((_RB, 1), lambda i: (i, 0)),
            pl.BlockSpec((1, D_HID), lambda i: (0, 0)),
            pl.BlockSpec((D_HID, 1), lambda i: (0, 0)),
        ],
        out_specs=pl.BlockSpec((_RB, 1), lambda i: (i, 0)),
        out_shape=jax.ShapeDtypeStruct((N, 1), jnp.float32),
    )(agg_parts, xws, dinv, b1, wz)


def _final_body(aggz_ref, zs_ref, dinv_ref, b2_ref, wlin_ref, blin_ref, o_ref):
    cval = jnp.dot(b2_ref[...], wlin_ref[...],
                   preferred_element_type=jnp.float32) + blin_ref[...]
    o_ref[...] = dinv_ref[...] * (aggz_ref[0] + aggz_ref[1] + zs_ref[...]) + cval


def _final_stage(aggz_parts, zs, dinv, b2, Wlin, blin):
    return pl.pallas_call(
        _final_body,
        grid=(_NG,),
        in_specs=[
            pl.BlockSpec((NC, _RB, 1), lambda i: (0, i, 0)),
            pl.BlockSpec((_RB, 1), lambda i: (i, 0)),
            pl.BlockSpec((_RB, 1), lambda i: (i, 0)),
            pl.BlockSpec((1, D_EMB), lambda i: (0, 0)),
            pl.BlockSpec((D_EMB, 1), lambda i: (0, 0)),
            pl.BlockSpec((1, 1), lambda i: (0, 0)),
        ],
        out_specs=pl.BlockSpec((_RB, 1), lambda i: (i, 0)),
        out_shape=jax.ShapeDtypeStruct((N, 1), jnp.float32),
    )(aggz_parts, zs, dinv, b2, Wlin, blin)


def kernel(x, edge_index, W1, b1, W2, b2, Wlin, blin):
    src = edge_index[0]
    dst = edge_index[1]
    zero_vec = jnp.zeros((STRIPE,), jnp.float32)
    zero_rows = jnp.zeros((RCH, D_HID), jnp.float32)

    deg_parts = _deg_counts(src, dst, zero_vec)
    xws, dinv, wz = _scale_stage(deg_parts.reshape(NC, N, 1), x, W1, W2, Wlin)
    agg_parts = _row_scatter(xws, src, dst, zero_rows)
    zs = _mid_stage(agg_parts.reshape(NC, N, D_HID), xws, dinv,
                    b1.reshape(1, D_HID), wz)
    aggz_parts = _scalar_scatter(zs.reshape(-1), src, dst, zero_vec)
    out = _final_stage(aggz_parts.reshape(NC, N, 1), zs, dinv,
                       b2.reshape(1, D_EMB), Wlin, blin.reshape(1, 1))
    return out.reshape(-1)


# speedup vs baseline: 36.8398x; 1.0326x over previous
import jax
import jax.numpy as jnp
from jax import lax
from jax.experimental import pallas as pl
from jax.experimental.pallas import tpu as pltpu
from jax.experimental.pallas import tpu_sc as plsc

N = 10000
E = 320000
D_IN = 128
D_HID = 128
D_EMB = 64

NC = 2
NS = 16
EPT = E // (NC * NS)
CH = 80
NCHUNK = EPT // CH
STRIPE = 624
TAIL = N - NS * STRIPE

_mesh = lambda: plsc.VectorSubcoreMesh(core_axis_name="c", subcore_axis_name="s")
_params = lambda: pltpu.CompilerParams(needs_layout_passes=False)


def _stage_indices(src_hbm, dst_hbm, src_all, dst_all, c, s):
    ebase = pl.multiple_of((c * NS + s) * EPT, 8)
    pltpu.sync_copy(src_hbm.at[pl.ds(ebase, EPT)], src_all)
    pltpu.sync_copy(dst_hbm.at[pl.ds(ebase, EPT)], dst_all)


def _zero_acc_1d(zero_hbm, buf_v, acc_sh, s):
    off0 = pl.multiple_of(s * STRIPE, 8)
    pltpu.sync_copy(zero_hbm, buf_v)
    pltpu.sync_copy(buf_v, acc_sh.at[pl.ds(off0, STRIPE)])

    @pl.when(s == NS - 1)
    def _zero_tail():
        pltpu.sync_copy(buf_v.at[pl.ds(0, TAIL)], acc_sh.at[pl.ds(N - TAIL, TAIL)])


def _readback_1d(acc_sh, buf_v, out_hbm, c, s):
    off0 = pl.multiple_of(s * STRIPE, 8)
    obase = pl.multiple_of(c * N, 8)
    pltpu.sync_copy(acc_sh.at[pl.ds(off0, STRIPE)], buf_v)
    pltpu.sync_copy(buf_v, out_hbm.at[pl.ds(obase + off0, STRIPE)])

    @pl.when(s == NS - 1)
    def _out_tail():
        pltpu.sync_copy(acc_sh.at[pl.ds(N - TAIL, TAIL)], buf_v.at[pl.ds(0, TAIL)])
        pltpu.sync_copy(buf_v.at[pl.ds(0, TAIL)],
                        out_hbm.at[pl.ds(obase + N - TAIL, TAIL)])


def _deg_body(src_hbm, dst_hbm, zero_hbm, out_hbm,
              src_all, dst_all, dstb, msg_v, buf_v, acc_sh):
    c = lax.axis_index("c")
    s = lax.axis_index("s")
    _stage_indices(src_hbm, dst_hbm, src_all, dst_all, c, s)
    _zero_acc_1d(zero_hbm, buf_v, acc_sh, s)
    for k in range(CH // 16):
        msg_v[pl.ds(k * 16, 16)] = jnp.ones((16,), jnp.float32)
    plsc.subcore_barrier()

    def chunk(i, carry):
        off = pl.multiple_of(i * CH, 8)
        for k in range(CH // 16):
            dstb[pl.ds(k * 16, 16)] = dst_all[pl.ds(off + k * 16, 16)]
        pltpu.sync_copy(msg_v, acc_sh.at[dstb], add=True)
        return carry

    lax.fori_loop(0, NCHUNK, chunk, 0)
    plsc.subcore_barrier()
    _readback_1d(acc_sh, buf_v, out_hbm, c, s)


def _deg_counts(src, dst, zero_vec):
    return pl.kernel(
        _deg_body,
        out_type=jax.ShapeDtypeStruct((NC * N,), jnp.float32),
        mesh=_mesh(),
        compiler_params=_params(),
        scratch_types=[
            pltpu.VMEM((EPT,), jnp.int32),
            pltpu.VMEM((EPT,), jnp.int32),
            pltpu.VMEM((CH,), jnp.int32),
            pltpu.VMEM((CH,), jnp.float32),
            pltpu.VMEM((STRIPE,), jnp.float32),
            pltpu.VMEM_SHARED((N,), jnp.float32),
        ],
    )(src, dst, zero_vec)


EPT2 = E // NS
NCHUNK2 = EPT2 // CH
FS = 312
HALF = N // NC


def _scalar_final_body(zs_hbm, fin_hbm, dinv_hbm, src_hbm, dst_hbm, zero_hbm,
                       out_hbm, vals_v, src_all, dst_all, dstb, msg_v, buf_v,
                       dinv_sv, fin_sv, outb_v, acc_sh):
    c = lax.axis_index("c")
    s = lax.axis_index("s")
    pltpu.sync_copy(zs_hbm, vals_v)
    ebase = pl.multiple_of(s * EPT2, 8)
    pltpu.sync_copy(src_hbm.at[pl.ds(ebase, EPT2)], src_all)
    pltpu.sync_copy(dst_hbm.at[pl.ds(ebase, EPT2)], dst_all)
    _zero_acc_1d(zero_hbm, buf_v, acc_sh, s)
    plsc.subcore_barrier()

    def chunk(i, carry):
        off = pl.multiple_of(i * CH, 8)
        for k in range(CH // 16):
            idx = src_all[pl.ds(off + k * 16, 16)]
            msg_v[pl.ds(k * 16, 16)] = plsc.load_gather(vals_v, [idx])
            dstb[pl.ds(k * 16, 16)] = dst_all[pl.ds(off + k * 16, 16)]
        pltpu.sync_copy(msg_v, acc_sh.at[dstb], add=True)
        return carry

    lax.fori_loop(0, NCHUNK2, chunk, 0)
    plsc.subcore_barrier()

    hbase = pl.multiple_of(c * HALF + s * FS, 8)
    pltpu.sync_copy(acc_sh.at[pl.ds(hbase, FS)], buf_v.at[pl.ds(0, FS)])
    pltpu.sync_copy(dinv_hbm.at[pl.ds(hbase, FS)], dinv_sv)
    pltpu.sync_copy(fin_hbm.at[pl.ds(hbase, FS)], fin_sv)
    for k in range(FS // 16):
        o = k * 16
        outb_v[pl.ds(o, 16)] = (dinv_sv[pl.ds(o, 16)] * buf_v[pl.ds(o, 16)]
                                + fin_sv[pl.ds(o, 16)])
    o = FS - 16
    outb_v[pl.ds(o, 16)] = (dinv_sv[pl.ds(o, 16)] * buf_v[pl.ds(o, 16)]
                            + fin_sv[pl.ds(o, 16)])
    pltpu.sync_copy(outb_v, out_hbm.at[pl.ds(hbase, FS)])

    @pl.when(s == NS - 1)
    def _fin_tail():
        tb = pl.multiple_of(c * HALF + HALF - 16, 8)
        pltpu.sync_copy(acc_sh.at[pl.ds(tb, 16)], buf_v.at[pl.ds(0, 16)])
        pltpu.sync_copy(dinv_hbm.at[pl.ds(tb, 16)], dinv_sv.at[pl.ds(0, 16)])
        pltpu.sync_copy(fin_hbm.at[pl.ds(tb, 16)], fin_sv.at[pl.ds(0, 16)])
        outb_v[pl.ds(0, 16)] = (dinv_sv[pl.ds(0, 16)] * buf_v[pl.ds(0, 16)]
                                + fin_sv[pl.ds(0, 16)])
        pltpu.sync_copy(outb_v.at[pl.ds(0, 16)], out_hbm.at[pl.ds(tb, 16)])


def _scalar_final(zs, fin, dinv, src, dst, zero_vec):
    return pl.kernel(
        _scalar_final_body,
        out_type=jax.ShapeDtypeStruct((N,), jnp.float32),
        mesh=_mesh(),
        compiler_params=_params(),
        scratch_types=[
            pltpu.VMEM((N,), jnp.float32),
            pltpu.VMEM((EPT2,), jnp.int32),
            pltpu.VMEM((EPT2,), jnp.int32),
            pltpu.VMEM((CH,), jnp.int32),
            pltpu.VMEM((CH,), jnp.float32),
            pltpu.VMEM((STRIPE,), jnp.float32),
            pltpu.VMEM((FS,), jnp.float32),
            pltpu.VMEM((FS,), jnp.float32),
            pltpu.VMEM((FS,), jnp.float32),
            pltpu.VMEM_SHARED((N,), jnp.float32),
        ],
    )(zs, fin, dinv, src, dst, zero_vec)


RCH = 128
RNCH = EPT // RCH
RTAIL = EPT - RNCH * RCH


def _row_scatter_body(rows_hbm, src_hbm, dst_hbm, zero_hbm, out_hbm,
                      src_all, dstb0, dstb1, dstbt, rows0, rows1,
                      acc_sh, sem0, sem1, semd0, semd1):
    c = lax.axis_index("c")
    s = lax.axis_index("s")
    ebase = pl.multiple_of((c * NS + s) * EPT, 8)
    pltpu.sync_copy(src_hbm.at[pl.ds(ebase, EPT)], src_all)
    off0 = pl.multiple_of(s * STRIPE, 8)
    pltpu.sync_copy(zero_hbm, rows0)
    for t in range(STRIPE // RCH):
        pltpu.sync_copy(rows0, acc_sh.at[pl.ds(off0 + t * RCH, RCH)])
    rem = STRIPE - (STRIPE // RCH) * RCH
    pltpu.sync_copy(rows0.at[pl.ds(0, rem)],
                    acc_sh.at[pl.ds(off0 + STRIPE - rem, rem)])

    @pl.when(s == NS - 1)
    def _zero_tail():
        pltpu.sync_copy(rows0.at[pl.ds(0, TAIL)], acc_sh.at[pl.ds(N - TAIL, TAIL)])

    plsc.subcore_barrier()

    def issue(rows_v, dstb, off, sem, semd):
        pltpu.async_copy(dst_hbm.at[pl.ds(ebase + off, RCH)], dstb, semd)
        pltpu.async_copy(rows_hbm.at[src_all.at[pl.ds(off, RCH)]], rows_v, sem)

    def drain(rows_v, dstb, off, sem, semd):
        pltpu.make_async_copy(dst_hbm.at[pl.ds(ebase + off, RCH)], dstb,
                              semd).wait()
        pltpu.make_async_copy(rows_hbm.at[src_all.at[pl.ds(off, RCH)]],
                              rows_v, sem).wait()
        pltpu.sync_copy(rows_v, acc_sh.at[dstb], add=True)

    issue(rows0, dstb0, 0, sem0, semd0)

    def pair(g, carry):
        o0 = pl.multiple_of(2 * g * RCH, 8)
        o1 = pl.multiple_of((2 * g + 1) * RCH, 8)
        o2 = pl.multiple_of((2 * g + 2) * RCH, 8)
        issue(rows1, dstb1, o1, sem1, semd1)
        drain(rows0, dstb0, o0, sem0, semd0)
        issue(rows0, dstb0, o2, sem0, semd0)
        drain(rows1, dstb1, o1, sem1, semd1)
        return carry

    lax.fori_loop(0, RNCH // 2 - 1, pair, 0)
    o76 = pl.multiple_of((RNCH - 2) * RCH, 8)
    o77 = pl.multiple_of((RNCH - 1) * RCH, 8)
    issue(rows1, dstb1, o77, sem1, semd1)
    drain(rows0, dstb0, o76, sem0, semd0)
    drain(rows1, dstb1, o77, sem1, semd1)
    ot = pl.multiple_of(RNCH * RCH, 8)
    pltpu.sync_copy(dst_hbm.at[pl.ds(ebase + ot, RTAIL)], dstbt)
    pltpu.async_copy(rows_hbm.at[src_all.at[pl.ds(ot, RTAIL)]],
                     rows0.at[pl.ds(0, RTAIL)], sem0).wait()
    pltpu.sync_copy(rows0.at[pl.ds(0, RTAIL)], acc_sh.at[dstbt], add=True)

    plsc.subcore_barrier()
    obase = pl.multiple_of(c * N, 8)
    for t in range(STRIPE // RCH):
        pltpu.sync_copy(acc_sh.at[pl.ds(off0 + t * RCH, RCH)], rows0)
        pltpu.sync_copy(rows0, out_hbm.at[pl.ds(obase + off0 + t * RCH, RCH)])
    pltpu.sync_copy(acc_sh.at[pl.ds(off0 + STRIPE - rem, rem)],
                    rows0.at[pl.ds(0, rem)])
    pltpu.sync_copy(rows0.at[pl.ds(0, rem)],
                    out_hbm.at[pl.ds(obase + off0 + STRIPE - rem, rem)])

    @pl.when(s == NS - 1)
    def _out_tail():
        pltpu.sync_copy(acc_sh.at[pl.ds(N - TAIL, TAIL)], rows1.at[pl.ds(0, TAIL)])
        pltpu.sync_copy(rows1.at[pl.ds(0, TAIL)],
                        out_hbm.at[pl.ds(obase + N - TAIL, TAIL)])


def _row_scatter(rows, src, dst, zero_rows):
    return pl.kernel(
        _row_scatter_body,
        out_type=jax.ShapeDtypeStruct((NC * N, D_HID), jnp.float32),
        mesh=_mesh(),
        compiler_params=_params(),
        scratch_types=[
            pltpu.VMEM((EPT,), jnp.int32),
            pltpu.VMEM((RCH,), jnp.int32),
            pltpu.VMEM((RCH,), jnp.int32),
            pltpu.VMEM((RTAIL,), jnp.int32),
            pltpu.VMEM((RCH, D_HID), jnp.float32),
            pltpu.VMEM((RCH, D_HID), jnp.float32),
            pltpu.VMEM_SHARED((N, D_HID), jnp.float32),
            pltpu.SemaphoreType.DMA,
            pltpu.SemaphoreType.DMA,
            pltpu.SemaphoreType.DMA,
            pltpu.SemaphoreType.DMA,
        ],
    )(rows, src, dst, zero_rows)


_RB = 400
_NG = N // _RB


def _scale_body(deg2_ref, x_ref, w1_ref, w2_ref, wlin_ref,
                xws_ref, dinv_ref, wz_ref):
    deg = deg2_ref[0] + deg2_ref[1] + 1.0
    dinv = lax.rsqrt(deg)
    dinv_ref[...] = dinv
    xw = jnp.dot(x_ref[...], w1_ref[...], preferred_element_type=jnp.float32)
    xws_ref[...] = dinv * xw
    wz_ref[...] = jnp.dot(w2_ref[...], wlin_ref[...],
                          preferred_element_type=jnp.float32)


def _scale_stage(deg_parts, x, W1, W2, Wlin):
    return pl.pallas_call(
        _scale_body,
        grid=(_NG,),
        in_specs=[
            pl.BlockSpec((NC, _RB, 1), lambda i: (0, i, 0)),
            pl.BlockSpec((_RB, D_IN), lambda i: (i, 0)),
            pl.BlockSpec((D_IN, D_HID), lambda i: (0, 0)),
            pl.BlockSpec((D_HID, D_EMB), lambda i: (0, 0)),
            pl.BlockSpec((D_EMB, 1), lambda i: (0, 0)),
        ],
        out_specs=[
            pl.BlockSpec((_RB, D_HID), lambda i: (i, 0)),
            pl.BlockSpec((_RB, 1), lambda i: (i, 0)),
            pl.BlockSpec((D_HID, 1), lambda i: (0, 0)),
        ],
        out_shape=[
            jax.ShapeDtypeStruct((N, D_HID), jnp.float32),
            jax.ShapeDtypeStruct((N, 1), jnp.float32),
            jax.ShapeDtypeStruct((D_HID, 1), jnp.float32),
        ],
    )(deg_parts, x, W1, W2, Wlin)


def _mid_body(agg_ref, xws_ref, dinv_ref, b1_ref, wz_ref, b2_ref, wlin_ref,
              blin_ref, zs_ref, fin_ref):
    dinv = dinv_ref[...]
    pre = dinv * (agg_ref[0] + agg_ref[1] + xws_ref[...]) + b1_ref[...]
    h = jnp.maximum(pre, 0.0)
    z = jnp.dot(h, wz_ref[...], preferred_element_type=jnp.float32)
    zs = dinv * z
    zs_ref[...] = zs
    cval = jnp.dot(b2_ref[...], wlin_ref[...],
                   preferred_element_type=jnp.float32) + blin_ref[...]
    fin_ref[...] = dinv * zs + cval


def _mid_stage(agg_parts, xws, dinv, b1, wz, b2, Wlin, blin):
    return pl.pallas_call(
        _mid_body,
        grid=(_NG,),
        in_specs=[
            pl.BlockSpec((NC, _RB, D_HID), lambda i: (0, i, 0)),
            pl.BlockSpec((_RB, D_HID), lambda i: (i, 0)),
            pl.BlockSpec((_RB, 1), lambda i: (i, 0)),
            pl.BlockSpec((1, D_HID), lambda i: (0, 0)),
            pl.BlockSpec((D_HID, 1), lambda i: (0, 0)),
            pl.BlockSpec((1, D_EMB), lambda i: (0, 0)),
            pl.BlockSpec((D_EMB, 1), lambda i: (0, 0)),
            pl.BlockSpec((1, 1), lambda i: (0, 0)),
        ],
        out_specs=[
            pl.BlockSpec((_RB, 1), lambda i: (i, 0)),
            pl.BlockSpec((_RB, 1), lambda i: (i, 0)),
        ],
        out_shape=[
            jax.ShapeDtypeStruct((N, 1), jnp.float32),
            jax.ShapeDtypeStruct((N, 1), jnp.float32),
        ],
    )(agg_parts, xws, dinv, b1, wz, b2, Wlin, blin)


def kernel(x, edge_index, W1, b1, W2, b2, Wlin, blin):
    src = edge_index[0]
    dst = edge_index[1]
    zero_vec = jnp.zeros((STRIPE,), jnp.float32)
    zero_rows = jnp.zeros((RCH, D_HID), jnp.float32)

    deg_parts = _deg_counts(src, dst, zero_vec)
    xws, dinv, wz = _scale_stage(deg_parts.reshape(NC, N, 1), x, W1, W2, Wlin)
    agg_parts = _row_scatter(xws, src, dst, zero_rows)
    zs, fin = _mid_stage(agg_parts.reshape(NC, N, D_HID), xws, dinv,
                         b1.reshape(1, D_HID), wz,
                         b2.reshape(1, D_EMB), Wlin, blin.reshape(1, 1))
    out = _scalar_final(zs.reshape(-1), fin.reshape(-1), dinv.reshape(-1),
                        src, dst, zero_vec)
    return out


# speedup vs baseline: 37.8607x; 1.0277x over previous
import jax
import jax.numpy as jnp
from jax import lax
from jax.experimental import pallas as pl
from jax.experimental.pallas import tpu as pltpu
from jax.experimental.pallas import tpu_sc as plsc

N = 10000
E = 320000
D_IN = 128
D_HID = 128
D_EMB = 64

NC = 2
NS = 16
EPT = E // (NC * NS)
CH = 80
NCHUNK = EPT // CH
STRIPE = 624
TAIL = N - NS * STRIPE

_mesh = lambda: plsc.VectorSubcoreMesh(core_axis_name="c", subcore_axis_name="s")
_params = lambda: pltpu.CompilerParams(needs_layout_passes=False)


def _stage_indices(src_hbm, dst_hbm, src_all, dst_all, c, s):
    ebase = pl.multiple_of((c * NS + s) * EPT, 8)
    pltpu.sync_copy(src_hbm.at[pl.ds(ebase, EPT)], src_all)
    pltpu.sync_copy(dst_hbm.at[pl.ds(ebase, EPT)], dst_all)


def _zero_acc_1d(zero_hbm, buf_v, acc_sh, s):
    off0 = pl.multiple_of(s * STRIPE, 8)
    pltpu.sync_copy(zero_hbm, buf_v)
    pltpu.sync_copy(buf_v, acc_sh.at[pl.ds(off0, STRIPE)])

    @pl.when(s == NS - 1)
    def _zero_tail():
        pltpu.sync_copy(buf_v.at[pl.ds(0, TAIL)], acc_sh.at[pl.ds(N - TAIL, TAIL)])


def _readback_1d(acc_sh, buf_v, out_hbm, c, s):
    off0 = pl.multiple_of(s * STRIPE, 8)
    obase = pl.multiple_of(c * N, 8)
    pltpu.sync_copy(acc_sh.at[pl.ds(off0, STRIPE)], buf_v)
    pltpu.sync_copy(buf_v, out_hbm.at[pl.ds(obase + off0, STRIPE)])

    @pl.when(s == NS - 1)
    def _out_tail():
        pltpu.sync_copy(acc_sh.at[pl.ds(N - TAIL, TAIL)], buf_v.at[pl.ds(0, TAIL)])
        pltpu.sync_copy(buf_v.at[pl.ds(0, TAIL)],
                        out_hbm.at[pl.ds(obase + N - TAIL, TAIL)])


def _deg_body(src_hbm, dst_hbm, zero_hbm, out_hbm,
              src_all, dst_all, dstb, msg_v, buf_v, acc_sh):
    c = lax.axis_index("c")
    s = lax.axis_index("s")
    _stage_indices(src_hbm, dst_hbm, src_all, dst_all, c, s)
    _zero_acc_1d(zero_hbm, buf_v, acc_sh, s)
    for k in range(CH // 16):
        msg_v[pl.ds(k * 16, 16)] = jnp.ones((16,), jnp.float32)
    plsc.subcore_barrier()

    def chunk(i, carry):
        off = pl.multiple_of(i * CH, 8)
        for k in range(CH // 16):
            dstb[pl.ds(k * 16, 16)] = dst_all[pl.ds(off + k * 16, 16)]
        pltpu.sync_copy(msg_v, acc_sh.at[dstb], add=True)
        return carry

    lax.fori_loop(0, NCHUNK, chunk, 0)
    plsc.subcore_barrier()
    _readback_1d(acc_sh, buf_v, out_hbm, c, s)


def _deg_counts(src, dst, zero_vec):
    return pl.kernel(
        _deg_body,
        out_type=jax.ShapeDtypeStruct((NC * N,), jnp.float32),
        mesh=_mesh(),
        compiler_params=_params(),
        scratch_types=[
            pltpu.VMEM((EPT,), jnp.int32),
            pltpu.VMEM((EPT,), jnp.int32),
            pltpu.VMEM((CH,), jnp.int32),
            pltpu.VMEM((CH,), jnp.float32),
            pltpu.VMEM((STRIPE,), jnp.float32),
            pltpu.VMEM_SHARED((N,), jnp.float32),
        ],
    )(src, dst, zero_vec)


EPT2 = E // NS
NCHUNK2 = EPT2 // CH
FS = 312
HALF = N // NC


def _scalar_final_body(zs_hbm, fin_hbm, dinv_hbm, src_hbm, dst_hbm, zero_hbm,
                       out_hbm, vals_v, src_all, dst_all, dstbs, msgs, buf_v,
                       dinv_sv, fin_sv, outb_v, acc_sh, csems):
    c = lax.axis_index("c")
    s = lax.axis_index("s")
    pltpu.sync_copy(zs_hbm, vals_v)
    ebase = pl.multiple_of(s * EPT2, 8)
    pltpu.sync_copy(src_hbm.at[pl.ds(ebase, EPT2)], src_all)
    pltpu.sync_copy(dst_hbm.at[pl.ds(ebase, EPT2)], dst_all)
    _zero_acc_1d(zero_hbm, buf_v, acc_sh, s)
    plsc.subcore_barrier()

    def fill_and_scatter(b, i):
        off = pl.multiple_of(i * CH, 8)
        for k in range(CH // 16):
            idx = src_all[pl.ds(off + k * 16, 16)]
            msgs[b][pl.ds(k * 16, 16)] = plsc.load_gather(vals_v, [idx])
            dstbs[b][pl.ds(k * 16, 16)] = dst_all[pl.ds(off + k * 16, 16)]
        pltpu.async_copy(msgs[b], acc_sh.at[dstbs[b]], csems[b], add=True)

    def wait_scatter(b):
        pltpu.make_async_copy(msgs[b], acc_sh.at[dstbs[b]], csems[b]).wait()

    fill_and_scatter(0, 0)
    fill_and_scatter(1, 1)

    def pair(g, carry):
        wait_scatter(0)
        fill_and_scatter(0, 2 * g + 2)
        wait_scatter(1)
        fill_and_scatter(1, 2 * g + 3)
        return carry

    lax.fori_loop(0, (NCHUNK2 - 2) // 2, pair, 0)
    wait_scatter(0)
    wait_scatter(1)
    plsc.subcore_barrier()

    hbase = pl.multiple_of(c * HALF + s * FS, 8)
    pltpu.sync_copy(acc_sh.at[pl.ds(hbase, FS)], buf_v.at[pl.ds(0, FS)])
    pltpu.sync_copy(dinv_hbm.at[pl.ds(hbase, FS)], dinv_sv)
    pltpu.sync_copy(fin_hbm.at[pl.ds(hbase, FS)], fin_sv)
    for k in range(FS // 16):
        o = k * 16
        outb_v[pl.ds(o, 16)] = (dinv_sv[pl.ds(o, 16)] * buf_v[pl.ds(o, 16)]
                                + fin_sv[pl.ds(o, 16)])
    o = FS - 16
    outb_v[pl.ds(o, 16)] = (dinv_sv[pl.ds(o, 16)] * buf_v[pl.ds(o, 16)]
                            + fin_sv[pl.ds(o, 16)])
    pltpu.sync_copy(outb_v, out_hbm.at[pl.ds(hbase, FS)])

    @pl.when(s == NS - 1)
    def _fin_tail():
        tb = pl.multiple_of(c * HALF + HALF - 16, 8)
        pltpu.sync_copy(acc_sh.at[pl.ds(tb, 16)], buf_v.at[pl.ds(0, 16)])
        pltpu.sync_copy(dinv_hbm.at[pl.ds(tb, 16)], dinv_sv.at[pl.ds(0, 16)])
        pltpu.sync_copy(fin_hbm.at[pl.ds(tb, 16)], fin_sv.at[pl.ds(0, 16)])
        outb_v[pl.ds(0, 16)] = (dinv_sv[pl.ds(0, 16)] * buf_v[pl.ds(0, 16)]
                                + fin_sv[pl.ds(0, 16)])
        pltpu.sync_copy(outb_v.at[pl.ds(0, 16)], out_hbm.at[pl.ds(tb, 16)])


def _scalar_final(zs, fin, dinv, src, dst, zero_vec):
    return pl.kernel(
        _scalar_final_body,
        out_type=jax.ShapeDtypeStruct((N,), jnp.float32),
        mesh=_mesh(),
        compiler_params=_params(),
        scratch_types=[
            pltpu.VMEM((N,), jnp.float32),
            pltpu.VMEM((EPT2,), jnp.int32),
            pltpu.VMEM((EPT2,), jnp.int32),
            [pltpu.VMEM((CH,), jnp.int32) for _ in range(2)],
            [pltpu.VMEM((CH,), jnp.float32) for _ in range(2)],
            pltpu.VMEM((STRIPE,), jnp.float32),
            pltpu.VMEM((FS,), jnp.float32),
            pltpu.VMEM((FS,), jnp.float32),
            pltpu.VMEM((FS,), jnp.float32),
            pltpu.VMEM_SHARED((N,), jnp.float32),
            [pltpu.SemaphoreType.DMA for _ in range(2)],
        ],
    )(zs, fin, dinv, src, dst, zero_vec)


RCH = 72
RNCH = EPT // RCH
RTAIL = EPT - RNCH * RCH
NBUF = 4


def _row_scatter_body(rows_hbm, src_hbm, dst_hbm, zero_hbm, out_hbm,
                      src_all, dstbs, dstbt, rows_bufs, acc_sh,
                      gsems, dsems, csems):
    c = lax.axis_index("c")
    s = lax.axis_index("s")
    ebase = pl.multiple_of((c * NS + s) * EPT, 8)
    pltpu.sync_copy(src_hbm.at[pl.ds(ebase, EPT)], src_all)
    rows0 = rows_bufs[0]
    off0 = pl.multiple_of(s * STRIPE, 8)
    pltpu.sync_copy(zero_hbm, rows0)
    for t in range(STRIPE // RCH):
        pltpu.sync_copy(rows0, acc_sh.at[pl.ds(off0 + t * RCH, RCH)])
    rem = STRIPE - (STRIPE // RCH) * RCH
    pltpu.sync_copy(rows0.at[pl.ds(0, rem)],
                    acc_sh.at[pl.ds(off0 + STRIPE - rem, rem)])

    @pl.when(s == NS - 1)
    def _zero_tail():
        pltpu.sync_copy(rows0.at[pl.ds(0, TAIL)], acc_sh.at[pl.ds(N - TAIL, TAIL)])

    plsc.subcore_barrier()

    def issue(b, off):
        pltpu.async_copy(dst_hbm.at[pl.ds(ebase + off, RCH)], dstbs[b],
                         dsems[b])
        pltpu.async_copy(rows_hbm.at[src_all.at[pl.ds(off, RCH)]],
                         rows_bufs[b], gsems[b])

    def process(b, off):
        pltpu.make_async_copy(dst_hbm.at[pl.ds(ebase + off, RCH)], dstbs[b],
                              dsems[b]).wait()
        pltpu.make_async_copy(rows_hbm.at[src_all.at[pl.ds(off, RCH)]],
                              rows_bufs[b], gsems[b]).wait()
        pltpu.async_copy(rows_bufs[b], acc_sh.at[dstbs[b]], csems[b],
                         add=True)

    def wait_scatter(b):
        pltpu.make_async_copy(rows_bufs[b], acc_sh.at[dstbs[b]],
                              csems[b]).wait()

    issue(0, pl.multiple_of(0, 8))
    issue(1, pl.multiple_of(RCH, 8))
    process(0, pl.multiple_of(0, 8))
    issue(2, pl.multiple_of(2 * RCH, 8))
    process(1, pl.multiple_of(RCH, 8))
    issue(3, pl.multiple_of(3 * RCH, 8))

    def quad(g, carry):
        for k in range(NBUF):
            t = 4 * g + 2 + k
            b = (2 + k) % NBUF
            b2 = (4 + k) % NBUF
            process(b, pl.multiple_of(t * RCH, 8))
            wait_scatter(b2)
            issue(b2, pl.multiple_of((t + 2) * RCH, 8))
        return carry

    lax.fori_loop(0, 33, quad, 0)
    for t in range(134, RNCH):
        b = t % NBUF
        b2 = (t + 2) % NBUF
        process(b, pl.multiple_of(t * RCH, 8))
        if t + 2 < RNCH:
            wait_scatter(b2)
            issue(b2, pl.multiple_of((t + 2) * RCH, 8))
    for b in range(NBUF):
        wait_scatter((RNCH - 4 + b) % NBUF)
    ot = pl.multiple_of(RNCH * RCH, 8)
    pltpu.sync_copy(dst_hbm.at[pl.ds(ebase + ot, RTAIL)], dstbt)
    pltpu.async_copy(rows_hbm.at[src_all.at[pl.ds(ot, RTAIL)]],
                     rows0.at[pl.ds(0, RTAIL)], gsems[0]).wait()
    pltpu.sync_copy(rows0.at[pl.ds(0, RTAIL)], acc_sh.at[dstbt], add=True)

    plsc.subcore_barrier()
    obase = pl.multiple_of(c * N, 8)
    for t in range(STRIPE // RCH):
        pltpu.sync_copy(acc_sh.at[pl.ds(off0 + t * RCH, RCH)], rows0)
        pltpu.sync_copy(rows0, out_hbm.at[pl.ds(obase + off0 + t * RCH, RCH)])
    pltpu.sync_copy(acc_sh.at[pl.ds(off0 + STRIPE - rem, rem)],
                    rows0.at[pl.ds(0, rem)])
    pltpu.sync_copy(rows0.at[pl.ds(0, rem)],
                    out_hbm.at[pl.ds(obase + off0 + STRIPE - rem, rem)])

    @pl.when(s == NS - 1)
    def _out_tail():
        rows1 = rows_bufs[1]
        pltpu.sync_copy(acc_sh.at[pl.ds(N - TAIL, TAIL)], rows1.at[pl.ds(0, TAIL)])
        pltpu.sync_copy(rows1.at[pl.ds(0, TAIL)],
                        out_hbm.at[pl.ds(obase + N - TAIL, TAIL)])


def _row_scatter(rows, src, dst, zero_rows):
    return pl.kernel(
        _row_scatter_body,
        out_type=jax.ShapeDtypeStruct((NC * N, D_HID), jnp.float32),
        mesh=_mesh(),
        compiler_params=_params(),
        scratch_types=[
            pltpu.VMEM((EPT,), jnp.int32),
            [pltpu.VMEM((RCH,), jnp.int32) for _ in range(NBUF)],
            pltpu.VMEM((RTAIL,), jnp.int32),
            [pltpu.VMEM((RCH, D_HID), jnp.float32) for _ in range(NBUF)],
            pltpu.VMEM_SHARED((N, D_HID), jnp.float32),
            [pltpu.SemaphoreType.DMA for _ in range(NBUF)],
            [pltpu.SemaphoreType.DMA for _ in range(NBUF)],
            [pltpu.SemaphoreType.DMA for _ in range(NBUF)],
        ],
    )(rows, src, dst, zero_rows)


_RB = 400
_NG = N // _RB


def _scale_body(deg2_ref, x_ref, w1_ref, w2_ref, wlin_ref,
                xws_ref, dinv_ref, wz_ref):
    deg = deg2_ref[0] + deg2_ref[1] + 1.0
    dinv = lax.rsqrt(deg)
    dinv_ref[...] = dinv
    xw = jnp.dot(x_ref[...], w1_ref[...], preferred_element_type=jnp.float32)
    xws_ref[...] = dinv * xw
    wz_ref[...] = jnp.dot(w2_ref[...], wlin_ref[...],
                          preferred_element_type=jnp.float32)


def _scale_stage(deg_parts, x, W1, W2, Wlin):
    return pl.pallas_call(
        _scale_body,
        grid=(_NG,),
        in_specs=[
            pl.BlockSpec((NC, _RB, 1), lambda i: (0, i, 0)),
            pl.BlockSpec((_RB, D_IN), lambda i: (i, 0)),
            pl.BlockSpec((D_IN, D_HID), lambda i: (0, 0)),
            pl.BlockSpec((D_HID, D_EMB), lambda i: (0, 0)),
            pl.BlockSpec((D_EMB, 1), lambda i: (0, 0)),
        ],
        out_specs=[
            pl.BlockSpec((_RB, D_HID), lambda i: (i, 0)),
            pl.BlockSpec((_RB, 1), lambda i: (i, 0)),
            pl.BlockSpec((D_HID, 1), lambda i: (0, 0)),
        ],
        out_shape=[
            jax.ShapeDtypeStruct((N, D_HID), jnp.float32),
            jax.ShapeDtypeStruct((N, 1), jnp.float32),
            jax.ShapeDtypeStruct((D_HID, 1), jnp.float32),
        ],
    )(deg_parts, x, W1, W2, Wlin)


def _mid_body(agg_ref, xws_ref, dinv_ref, b1_ref, wz_ref, b2_ref, wlin_ref,
              blin_ref, zs_ref, fin_ref):
    dinv = dinv_ref[...]
    pre = dinv * (agg_ref[0] + agg_ref[1] + xws_ref[...]) + b1_ref[...]
    h = jnp.maximum(pre, 0.0)
    z = jnp.dot(h, wz_ref[...], preferred_element_type=jnp.float32)
    zs = dinv * z
    zs_ref[...] = zs
    cval = jnp.dot(b2_ref[...], wlin_ref[...],
                   preferred_element_type=jnp.float32) + blin_ref[...]
    fin_ref[...] = dinv * zs + cval


def _mid_stage(agg_parts, xws, dinv, b1, wz, b2, Wlin, blin):
    return pl.pallas_call(
        _mid_body,
        grid=(_NG,),
        in_specs=[
            pl.BlockSpec((NC, _RB, D_HID), lambda i: (0, i, 0)),
            pl.BlockSpec((_RB, D_HID), lambda i: (i, 0)),
            pl.BlockSpec((_RB, 1), lambda i: (i, 0)),
            pl.BlockSpec((1, D_HID), lambda i: (0, 0)),
            pl.BlockSpec((D_HID, 1), lambda i: (0, 0)),
            pl.BlockSpec((1, D_EMB), lambda i: (0, 0)),
            pl.BlockSpec((D_EMB, 1), lambda i: (0, 0)),
            pl.BlockSpec((1, 1), lambda i: (0, 0)),
        ],
        out_specs=[
            pl.BlockSpec((_RB, 1), lambda i: (i, 0)),
            pl.BlockSpec((_RB, 1), lambda i: (i, 0)),
        ],
        out_shape=[
            jax.ShapeDtypeStruct((N, 1), jnp.float32),
            jax.ShapeDtypeStruct((N, 1), jnp.float32),
        ],
    )(agg_parts, xws, dinv, b1, wz, b2, Wlin, blin)


def kernel(x, edge_index, W1, b1, W2, b2, Wlin, blin):
    src = edge_index[0]
    dst = edge_index[1]
    zero_vec = jnp.zeros((STRIPE,), jnp.float32)
    zero_rows = jnp.zeros((RCH, D_HID), jnp.float32)

    deg_parts = _deg_counts(src, dst, zero_vec)
    xws, dinv, wz = _scale_stage(deg_parts.reshape(NC, N, 1), x, W1, W2, Wlin)
    agg_parts = _row_scatter(xws, src, dst, zero_rows)
    zs, fin = _mid_stage(agg_parts.reshape(NC, N, D_HID), xws, dinv,
                         b1.reshape(1, D_HID), wz,
                         b2.reshape(1, D_EMB), Wlin, blin.reshape(1, 1))
    out = _scalar_final(zs.reshape(-1), fin.reshape(-1), dinv.reshape(-1),
                        src, dst, zero_vec)
    return out


# speedup vs baseline: 39.4875x; 1.0430x over previous
import jax
import jax.numpy as jnp
from jax import lax
from jax.experimental import pallas as pl
from jax.experimental.pallas import tpu as pltpu
from jax.experimental.pallas import tpu_sc as plsc

N = 10000
E = 320000
D_IN = 128
D_HID = 128
D_EMB = 64

NC = 2
NS = 16
EPT = E // (NC * NS)
CH = 80
NCHUNK = EPT // CH
STRIPE = 624
TAIL = N - NS * STRIPE

_mesh = lambda: plsc.VectorSubcoreMesh(core_axis_name="c", subcore_axis_name="s")
_params = lambda: pltpu.CompilerParams(needs_layout_passes=False)


def _stage_indices(src_hbm, dst_hbm, src_all, dst_all, c, s):
    ebase = pl.multiple_of((c * NS + s) * EPT, 8)
    pltpu.sync_copy(src_hbm.at[pl.ds(ebase, EPT)], src_all)
    pltpu.sync_copy(dst_hbm.at[pl.ds(ebase, EPT)], dst_all)


def _zero_acc_1d(zero_hbm, buf_v, acc_sh, s):
    off0 = pl.multiple_of(s * STRIPE, 8)
    pltpu.sync_copy(zero_hbm, buf_v)
    pltpu.sync_copy(buf_v, acc_sh.at[pl.ds(off0, STRIPE)])

    @pl.when(s == NS - 1)
    def _zero_tail():
        pltpu.sync_copy(buf_v.at[pl.ds(0, TAIL)], acc_sh.at[pl.ds(N - TAIL, TAIL)])


def _readback_1d(acc_sh, buf_v, out_hbm, c, s):
    off0 = pl.multiple_of(s * STRIPE, 8)
    obase = pl.multiple_of(c * N, 8)
    pltpu.sync_copy(acc_sh.at[pl.ds(off0, STRIPE)], buf_v)
    pltpu.sync_copy(buf_v, out_hbm.at[pl.ds(obase + off0, STRIPE)])

    @pl.when(s == NS - 1)
    def _out_tail():
        pltpu.sync_copy(acc_sh.at[pl.ds(N - TAIL, TAIL)], buf_v.at[pl.ds(0, TAIL)])
        pltpu.sync_copy(buf_v.at[pl.ds(0, TAIL)],
                        out_hbm.at[pl.ds(obase + N - TAIL, TAIL)])


def _deg_body(src_hbm, dst_hbm, zero_hbm, out_hbm,
              src_all, dst_all, dstb, msg_v, buf_v, acc_sh):
    c = lax.axis_index("c")
    s = lax.axis_index("s")
    _stage_indices(src_hbm, dst_hbm, src_all, dst_all, c, s)
    _zero_acc_1d(zero_hbm, buf_v, acc_sh, s)
    for k in range(CH // 16):
        msg_v[pl.ds(k * 16, 16)] = jnp.ones((16,), jnp.float32)
    plsc.subcore_barrier()

    def chunk(i, carry):
        off = pl.multiple_of(i * CH, 8)
        for k in range(CH // 16):
            dstb[pl.ds(k * 16, 16)] = dst_all[pl.ds(off + k * 16, 16)]
        pltpu.sync_copy(msg_v, acc_sh.at[dstb], add=True)
        return carry

    lax.fori_loop(0, NCHUNK, chunk, 0)
    plsc.subcore_barrier()
    _readback_1d(acc_sh, buf_v, out_hbm, c, s)


def _deg_counts(src, dst, zero_vec):
    return pl.kernel(
        _deg_body,
        out_type=jax.ShapeDtypeStruct((NC * N,), jnp.float32),
        mesh=_mesh(),
        compiler_params=_params(),
        scratch_types=[
            pltpu.VMEM((EPT,), jnp.int32),
            pltpu.VMEM((EPT,), jnp.int32),
            pltpu.VMEM((CH,), jnp.int32),
            pltpu.VMEM((CH,), jnp.float32),
            pltpu.VMEM((STRIPE,), jnp.float32),
            pltpu.VMEM_SHARED((N,), jnp.float32),
        ],
    )(src, dst, zero_vec)


EPT2 = E // NS
NCHUNK2 = EPT2 // CH
FS = 312
HALF = N // NC


def _scalar_final_body(zs_hbm, fin_hbm, dinv_hbm, src_hbm, dst_hbm, zero_hbm,
                       out_hbm, vals_v, src_all, dst_all, dstbs, msgs, buf_v,
                       dinv_sv, fin_sv, outb_v, acc_sh, csems):
    c = lax.axis_index("c")
    s = lax.axis_index("s")
    pltpu.sync_copy(zs_hbm, vals_v)
    ebase = pl.multiple_of(s * EPT2, 8)
    pltpu.sync_copy(src_hbm.at[pl.ds(ebase, EPT2)], src_all)
    pltpu.sync_copy(dst_hbm.at[pl.ds(ebase, EPT2)], dst_all)
    _zero_acc_1d(zero_hbm, buf_v, acc_sh, s)
    plsc.subcore_barrier()

    def fill_and_scatter(b, i):
        off = pl.multiple_of(i * CH, 8)
        for k in range(CH // 16):
            idx = src_all[pl.ds(off + k * 16, 16)]
            msgs[b][pl.ds(k * 16, 16)] = plsc.load_gather(vals_v, [idx])
            dstbs[b][pl.ds(k * 16, 16)] = dst_all[pl.ds(off + k * 16, 16)]
        pltpu.async_copy(msgs[b], acc_sh.at[dstbs[b]], csems[b], add=True)

    def wait_scatter(b):
        pltpu.make_async_copy(msgs[b], acc_sh.at[dstbs[b]], csems[b]).wait()

    fill_and_scatter(0, 0)
    fill_and_scatter(1, 1)

    def pair(g, carry):
        wait_scatter(0)
        fill_and_scatter(0, 2 * g + 2)
        wait_scatter(1)
        fill_and_scatter(1, 2 * g + 3)
        return carry

    lax.fori_loop(0, (NCHUNK2 - 2) // 2, pair, 0)
    wait_scatter(0)
    wait_scatter(1)
    plsc.subcore_barrier()

    hbase = pl.multiple_of(c * HALF + s * FS, 8)
    pltpu.sync_copy(acc_sh.at[pl.ds(hbase, FS)], buf_v.at[pl.ds(0, FS)])
    pltpu.sync_copy(dinv_hbm.at[pl.ds(hbase, FS)], dinv_sv)
    pltpu.sync_copy(fin_hbm.at[pl.ds(hbase, FS)], fin_sv)
    for k in range(FS // 16):
        o = k * 16
        outb_v[pl.ds(o, 16)] = (dinv_sv[pl.ds(o, 16)] * buf_v[pl.ds(o, 16)]
                                + fin_sv[pl.ds(o, 16)])
    o = FS - 16
    outb_v[pl.ds(o, 16)] = (dinv_sv[pl.ds(o, 16)] * buf_v[pl.ds(o, 16)]
                            + fin_sv[pl.ds(o, 16)])
    pltpu.sync_copy(outb_v, out_hbm.at[pl.ds(hbase, FS)])

    @pl.when(s == NS - 1)
    def _fin_tail():
        tb = pl.multiple_of(c * HALF + HALF - 16, 8)
        pltpu.sync_copy(acc_sh.at[pl.ds(tb, 16)], buf_v.at[pl.ds(0, 16)])
        pltpu.sync_copy(dinv_hbm.at[pl.ds(tb, 16)], dinv_sv.at[pl.ds(0, 16)])
        pltpu.sync_copy(fin_hbm.at[pl.ds(tb, 16)], fin_sv.at[pl.ds(0, 16)])
        outb_v[pl.ds(0, 16)] = (dinv_sv[pl.ds(0, 16)] * buf_v[pl.ds(0, 16)]
                                + fin_sv[pl.ds(0, 16)])
        pltpu.sync_copy(outb_v.at[pl.ds(0, 16)], out_hbm.at[pl.ds(tb, 16)])


def _scalar_final(zs, fin, dinv, src, dst, zero_vec):
    return pl.kernel(
        _scalar_final_body,
        out_type=jax.ShapeDtypeStruct((N,), jnp.float32),
        mesh=_mesh(),
        compiler_params=_params(),
        scratch_types=[
            pltpu.VMEM((N,), jnp.float32),
            pltpu.VMEM((EPT2,), jnp.int32),
            pltpu.VMEM((EPT2,), jnp.int32),
            [pltpu.VMEM((CH,), jnp.int32) for _ in range(2)],
            [pltpu.VMEM((CH,), jnp.float32) for _ in range(2)],
            pltpu.VMEM((STRIPE,), jnp.float32),
            pltpu.VMEM((FS,), jnp.float32),
            pltpu.VMEM((FS,), jnp.float32),
            pltpu.VMEM((FS,), jnp.float32),
            pltpu.VMEM_SHARED((N,), jnp.float32),
            [pltpu.SemaphoreType.DMA for _ in range(2)],
        ],
    )(zs, fin, dinv, src, dst, zero_vec)


RCH = 128
RNCH = EPT // RCH
RTAIL = EPT - RNCH * RCH
NBUF = 2


def _row_scatter_body(rows_hbm, src_hbm, dst_hbm, zero_hbm, out_hbm,
                      src_all, dstbs, dstbt, rows_bufs, acc_sh,
                      gsems, dsems, csems):
    c = lax.axis_index("c")
    s = lax.axis_index("s")
    ebase = pl.multiple_of((c * NS + s) * EPT, 8)
    pltpu.sync_copy(src_hbm.at[pl.ds(ebase, EPT)], src_all)
    rows0 = rows_bufs[0]
    off0 = pl.multiple_of(s * STRIPE, 8)
    pltpu.sync_copy(zero_hbm, rows0)
    for t in range(STRIPE // RCH):
        pltpu.sync_copy(rows0, acc_sh.at[pl.ds(off0 + t * RCH, RCH)])
    rem = STRIPE - (STRIPE // RCH) * RCH
    pltpu.sync_copy(rows0.at[pl.ds(0, rem)],
                    acc_sh.at[pl.ds(off0 + STRIPE - rem, rem)])

    @pl.when(s == NS - 1)
    def _zero_tail():
        pltpu.sync_copy(rows0.at[pl.ds(0, TAIL)], acc_sh.at[pl.ds(N - TAIL, TAIL)])

    plsc.subcore_barrier()

    def issue(b, off):
        pltpu.async_copy(dst_hbm.at[pl.ds(ebase + off, RCH)], dstbs[b],
                         dsems[b])
        pltpu.async_copy(rows_hbm.at[src_all.at[pl.ds(off, RCH)]],
                         rows_bufs[b], gsems[b])

    def process(b, off):
        pltpu.make_async_copy(dst_hbm.at[pl.ds(ebase + off, RCH)], dstbs[b],
                              dsems[b]).wait()
        pltpu.make_async_copy(rows_hbm.at[src_all.at[pl.ds(off, RCH)]],
                              rows_bufs[b], gsems[b]).wait()
        pltpu.async_copy(rows_bufs[b], acc_sh.at[dstbs[b]], csems[b],
                         add=True)

    def wait_scatter(b):
        pltpu.make_async_copy(rows_bufs[b], acc_sh.at[dstbs[b]],
                              csems[b]).wait()

    issue(0, pl.multiple_of(0, 8))

    def pair(g, carry):
        o0 = pl.multiple_of(2 * g * RCH, 8)
        o1 = pl.multiple_of((2 * g + 1) * RCH, 8)
        o2 = pl.multiple_of((2 * g + 2) * RCH, 8)
        issue(1, o1)
        process(0, o0)
        wait_scatter(0)
        issue(0, o2)
        process(1, o1)
        wait_scatter(1)
        return carry

    lax.fori_loop(0, RNCH // 2 - 1, pair, 0)
    o76 = pl.multiple_of((RNCH - 2) * RCH, 8)
    o77 = pl.multiple_of((RNCH - 1) * RCH, 8)
    issue(1, o77)
    process(0, o76)
    wait_scatter(0)
    process(1, o77)
    wait_scatter(1)
    ot = pl.multiple_of(RNCH * RCH, 8)
    pltpu.sync_copy(dst_hbm.at[pl.ds(ebase + ot, RTAIL)], dstbt)
    pltpu.async_copy(rows_hbm.at[src_all.at[pl.ds(ot, RTAIL)]],
                     rows0.at[pl.ds(0, RTAIL)], gsems[0]).wait()
    pltpu.sync_copy(rows0.at[pl.ds(0, RTAIL)], acc_sh.at[dstbt], add=True)

    plsc.subcore_barrier()
    obase = pl.multiple_of(c * N, 8)
    for t in range(STRIPE // RCH):
        pltpu.sync_copy(acc_sh.at[pl.ds(off0 + t * RCH, RCH)], rows0)
        pltpu.sync_copy(rows0, out_hbm.at[pl.ds(obase + off0 + t * RCH, RCH)])
    pltpu.sync_copy(acc_sh.at[pl.ds(off0 + STRIPE - rem, rem)],
                    rows0.at[pl.ds(0, rem)])
    pltpu.sync_copy(rows0.at[pl.ds(0, rem)],
                    out_hbm.at[pl.ds(obase + off0 + STRIPE - rem, rem)])

    @pl.when(s == NS - 1)
    def _out_tail():
        rows1 = rows_bufs[1]
        pltpu.sync_copy(acc_sh.at[pl.ds(N - TAIL, TAIL)], rows1.at[pl.ds(0, TAIL)])
        pltpu.sync_copy(rows1.at[pl.ds(0, TAIL)],
                        out_hbm.at[pl.ds(obase + N - TAIL, TAIL)])


def _row_scatter(rows, src, dst, zero_rows):
    return pl.kernel(
        _row_scatter_body,
        out_type=jax.ShapeDtypeStruct((NC * N, D_HID), jnp.float32),
        mesh=_mesh(),
        compiler_params=_params(),
        scratch_types=[
            pltpu.VMEM((EPT,), jnp.int32),
            [pltpu.VMEM((RCH,), jnp.int32) for _ in range(NBUF)],
            pltpu.VMEM((RTAIL,), jnp.int32),
            [pltpu.VMEM((RCH, D_HID), jnp.float32) for _ in range(NBUF)],
            pltpu.VMEM_SHARED((N, D_HID), jnp.float32),
            [pltpu.SemaphoreType.DMA for _ in range(NBUF)],
            [pltpu.SemaphoreType.DMA for _ in range(NBUF)],
            [pltpu.SemaphoreType.DMA for _ in range(NBUF)],
        ],
    )(rows, src, dst, zero_rows)


_RB = 400
_NG = N // _RB


def _scale_body(deg2_ref, x_ref, w1_ref, w2_ref, wlin_ref,
                xws_ref, dinv_ref, wz_ref):
    deg = deg2_ref[0] + deg2_ref[1] + 1.0
    dinv = lax.rsqrt(deg)
    dinv_ref[...] = dinv
    xw = jnp.dot(x_ref[...], w1_ref[...], preferred_element_type=jnp.float32)
    xws_ref[...] = dinv * xw
    wz_ref[...] = jnp.dot(w2_ref[...], wlin_ref[...],
                          preferred_element_type=jnp.float32)


def _scale_stage(deg_parts, x, W1, W2, Wlin):
    return pl.pallas_call(
        _scale_body,
        grid=(_NG,),
        in_specs=[
            pl.BlockSpec((NC, _RB, 1), lambda i: (0, i, 0)),
            pl.BlockSpec((_RB, D_IN), lambda i: (i, 0)),
            pl.BlockSpec((D_IN, D_HID), lambda i: (0, 0)),
            pl.BlockSpec((D_HID, D_EMB), lambda i: (0, 0)),
            pl.BlockSpec((D_EMB, 1), lambda i: (0, 0)),
        ],
        out_specs=[
            pl.BlockSpec((_RB, D_HID), lambda i: (i, 0)),
            pl.BlockSpec((_RB, 1), lambda i: (i, 0)),
            pl.BlockSpec((D_HID, 1), lambda i: (0, 0)),
        ],
        out_shape=[
            jax.ShapeDtypeStruct((N, D_HID), jnp.float32),
            jax.ShapeDtypeStruct((N, 1), jnp.float32),
            jax.ShapeDtypeStruct((D_HID, 1), jnp.float32),
        ],
    )(deg_parts, x, W1, W2, Wlin)


def _mid_body(agg_ref, xws_ref, dinv_ref, b1_ref, wz_ref, b2_ref, wlin_ref,
              blin_ref, zs_ref, fin_ref):
    dinv = dinv_ref[...]
    pre = dinv * (agg_ref[0] + agg_ref[1] + xws_ref[...]) + b1_ref[...]
    h = jnp.maximum(pre, 0.0)
    z = jnp.dot(h, wz_ref[...], preferred_element_type=jnp.float32)
    zs = dinv * z
    zs_ref[...] = zs
    cval = jnp.dot(b2_ref[...], wlin_ref[...],
                   preferred_element_type=jnp.float32) + blin_ref[...]
    fin_ref[...] = dinv * zs + cval


def _mid_stage(agg_parts, xws, dinv, b1, wz, b2, Wlin, blin):
    return pl.pallas_call(
        _mid_body,
        grid=(_NG,),
        in_specs=[
            pl.BlockSpec((NC, _RB, D_HID), lambda i: (0, i, 0)),
            pl.BlockSpec((_RB, D_HID), lambda i: (i, 0)),
            pl.BlockSpec((_RB, 1), lambda i: (i, 0)),
            pl.BlockSpec((1, D_HID), lambda i: (0, 0)),
            pl.BlockSpec((D_HID, 1), lambda i: (0, 0)),
            pl.BlockSpec((1, D_EMB), lambda i: (0, 0)),
            pl.BlockSpec((D_EMB, 1), lambda i: (0, 0)),
            pl.BlockSpec((1, 1), lambda i: (0, 0)),
        ],
        out_specs=[
            pl.BlockSpec((_RB, 1), lambda i: (i, 0)),
            pl.BlockSpec((_RB, 1), lambda i: (i, 0)),
        ],
        out_shape=[
            jax.ShapeDtypeStruct((N, 1), jnp.float32),
            jax.ShapeDtypeStruct((N, 1), jnp.float32),
        ],
    )(agg_parts, xws, dinv, b1, wz, b2, Wlin, blin)


def kernel(x, edge_index, W1, b1, W2, b2, Wlin, blin):
    src = edge_index[0]
    dst = edge_index[1]
    zero_vec = jnp.zeros((STRIPE,), jnp.float32)
    zero_rows = jnp.zeros((RCH, D_HID), jnp.float32)

    deg_parts = _deg_counts(src, dst, zero_vec)
    xws, dinv, wz = _scale_stage(deg_parts.reshape(NC, N, 1), x, W1, W2, Wlin)
    agg_parts = _row_scatter(xws, src, dst, zero_rows)
    zs, fin = _mid_stage(agg_parts.reshape(NC, N, D_HID), xws, dinv,
                         b1.reshape(1, D_HID), wz,
                         b2.reshape(1, D_EMB), Wlin, blin.reshape(1, 1))
    out = _scalar_final(zs.reshape(-1), fin.reshape(-1), dinv.reshape(-1),
                        src, dst, zero_vec)
    return out


# speedup vs baseline: 41.2500x; 1.0446x over previous
import jax
import jax.numpy as jnp
from jax import lax
from jax.experimental import pallas as pl
from jax.experimental.pallas import tpu as pltpu
from jax.experimental.pallas import tpu_sc as plsc

N = 10000
E = 320000
D_IN = 128
D_HID = 128
D_EMB = 64

NC = 2
NS = 16
EPT = E // (NC * NS)
CH = 80
NCHUNK = EPT // CH
STRIPE = 624
TAIL = N - NS * STRIPE

_mesh = lambda: plsc.VectorSubcoreMesh(core_axis_name="c", subcore_axis_name="s")
_params = lambda: pltpu.CompilerParams(needs_layout_passes=False)


def _zero_acc_1d(zero_hbm, buf_v, acc_sh, s):
    off0 = pl.multiple_of(s * STRIPE, 8)
    pltpu.sync_copy(zero_hbm, buf_v)
    pltpu.sync_copy(buf_v, acc_sh.at[pl.ds(off0, STRIPE)])

    @pl.when(s == NS - 1)
    def _zero_tail():
        pltpu.sync_copy(buf_v.at[pl.ds(0, TAIL)], acc_sh.at[pl.ds(N - TAIL, TAIL)])


def _readback_1d(acc_sh, buf_v, out_hbm, c, s):
    off0 = pl.multiple_of(s * STRIPE, 8)
    obase = pl.multiple_of(c * N, 8)
    pltpu.sync_copy(acc_sh.at[pl.ds(off0, STRIPE)], buf_v)
    pltpu.sync_copy(buf_v, out_hbm.at[pl.ds(obase + off0, STRIPE)])

    @pl.when(s == NS - 1)
    def _out_tail():
        pltpu.sync_copy(acc_sh.at[pl.ds(N - TAIL, TAIL)], buf_v.at[pl.ds(0, TAIL)])
        pltpu.sync_copy(buf_v.at[pl.ds(0, TAIL)],
                        out_hbm.at[pl.ds(obase + N - TAIL, TAIL)])


DCH = 128
DNCH = EPT // DCH
DTAIL = EPT - DNCH * DCH


def _deg_body(dst_hbm, zero_hbm, out_hbm,
              dst_all, dstbs, dstbt, ones_v, buf_v, acc_sh, csems):
    c = lax.axis_index("c")
    s = lax.axis_index("s")
    ebase = pl.multiple_of((c * NS + s) * EPT, 8)
    pltpu.sync_copy(dst_hbm.at[pl.ds(ebase, EPT)], dst_all)
    _zero_acc_1d(zero_hbm, buf_v, acc_sh, s)
    for k in range(DCH // 16):
        ones_v[pl.ds(k * 16, 16)] = jnp.ones((16,), jnp.float32)
    plsc.subcore_barrier()

    def fill_and_scatter(b, i):
        off = pl.multiple_of(i * DCH, 8)
        for k in range(DCH // 16):
            dstbs[b][pl.ds(k * 16, 16)] = dst_all[pl.ds(off + k * 16, 16)]
        pltpu.async_copy(ones_v, acc_sh.at[dstbs[b]], csems[b], add=True)

    def wait_scatter(b):
        pltpu.make_async_copy(ones_v, acc_sh.at[dstbs[b]], csems[b]).wait()

    fill_and_scatter(0, 0)
    fill_and_scatter(1, 1)

    def pair(g, carry):
        wait_scatter(0)
        fill_and_scatter(0, 2 * g + 2)
        wait_scatter(1)
        fill_and_scatter(1, 2 * g + 3)
        return carry

    lax.fori_loop(0, (DNCH - 2) // 2, pair, 0)
    wait_scatter(0)
    wait_scatter(1)
    ot = pl.multiple_of(DNCH * DCH, 8)
    for k in range(DTAIL // 16):
        dstbt[pl.ds(k * 16, 16)] = dst_all[pl.ds(ot + k * 16, 16)]
    pltpu.sync_copy(ones_v.at[pl.ds(0, DTAIL)], acc_sh.at[dstbt], add=True)
    plsc.subcore_barrier()
    _readback_1d(acc_sh, buf_v, out_hbm, c, s)


def _deg_counts(dst, zero_vec):
    return pl.kernel(
        _deg_body,
        out_type=jax.ShapeDtypeStruct((NC * N,), jnp.float32),
        mesh=_mesh(),
        compiler_params=_params(),
        scratch_types=[
            pltpu.VMEM((EPT,), jnp.int32),
            [pltpu.VMEM((DCH,), jnp.int32) for _ in range(2)],
            pltpu.VMEM((DTAIL,), jnp.int32),
            pltpu.VMEM((DCH,), jnp.float32),
            pltpu.VMEM((STRIPE,), jnp.float32),
            pltpu.VMEM_SHARED((N,), jnp.float32),
            [pltpu.SemaphoreType.DMA for _ in range(2)],
        ],
    )(dst, zero_vec)


EPT2 = E // NS
SCH = 128
SNCH = EPT2 // SCH
STAIL = EPT2 - SNCH * SCH
FS = 312
HALF = N // NC


def _scalar_final_body(zs_hbm, fin_hbm, dinv_hbm, src_hbm, dst_hbm, zero_hbm,
                       out_hbm, vals_v, src_all, dst_all, dstbs, dstbt, msgs,
                       buf_v, dinv_sv, fin_sv, outb_v, acc_sh, csems):
    c = lax.axis_index("c")
    s = lax.axis_index("s")
    pltpu.sync_copy(zs_hbm, vals_v)
    ebase = pl.multiple_of(s * EPT2, 8)
    pltpu.sync_copy(src_hbm.at[pl.ds(ebase, EPT2)], src_all)
    pltpu.sync_copy(dst_hbm.at[pl.ds(ebase, EPT2)], dst_all)
    _zero_acc_1d(zero_hbm, buf_v, acc_sh, s)
    plsc.subcore_barrier()

    def fill_and_scatter(b, i):
        off = pl.multiple_of(i * SCH, 8)
        for k in range(SCH // 16):
            idx = src_all[pl.ds(off + k * 16, 16)]
            msgs[b][pl.ds(k * 16, 16)] = plsc.load_gather(vals_v, [idx])
            dstbs[b][pl.ds(k * 16, 16)] = dst_all[pl.ds(off + k * 16, 16)]
        pltpu.async_copy(msgs[b], acc_sh.at[dstbs[b]], csems[b], add=True)

    def wait_scatter(b):
        pltpu.make_async_copy(msgs[b], acc_sh.at[dstbs[b]], csems[b]).wait()

    fill_and_scatter(0, 0)
    fill_and_scatter(1, 1)

    def pair(g, carry):
        wait_scatter(0)
        fill_and_scatter(0, 2 * g + 2)
        wait_scatter(1)
        fill_and_scatter(1, 2 * g + 3)
        return carry

    lax.fori_loop(0, (SNCH - 2) // 2, pair, 0)
    wait_scatter(0)
    wait_scatter(1)
    ot = pl.multiple_of(SNCH * SCH, 8)
    for k in range(STAIL // 16):
        idx = src_all[pl.ds(ot + k * 16, 16)]
        msgs[0][pl.ds(k * 16, 16)] = plsc.load_gather(vals_v, [idx])
        dstbt[pl.ds(k * 16, 16)] = dst_all[pl.ds(ot + k * 16, 16)]
    pltpu.sync_copy(msgs[0].at[pl.ds(0, STAIL)], acc_sh.at[dstbt], add=True)
    plsc.subcore_barrier()

    hbase = pl.multiple_of(c * HALF + s * FS, 8)
    pltpu.sync_copy(acc_sh.at[pl.ds(hbase, FS)], buf_v.at[pl.ds(0, FS)])
    pltpu.sync_copy(dinv_hbm.at[pl.ds(hbase, FS)], dinv_sv)
    pltpu.sync_copy(fin_hbm.at[pl.ds(hbase, FS)], fin_sv)
    for k in range(FS // 16):
        o = k * 16
        outb_v[pl.ds(o, 16)] = (dinv_sv[pl.ds(o, 16)] * buf_v[pl.ds(o, 16)]
                                + fin_sv[pl.ds(o, 16)])
    o = FS - 16
    outb_v[pl.ds(o, 16)] = (dinv_sv[pl.ds(o, 16)] * buf_v[pl.ds(o, 16)]
                            + fin_sv[pl.ds(o, 16)])
    pltpu.sync_copy(outb_v, out_hbm.at[pl.ds(hbase, FS)])

    @pl.when(s == NS - 1)
    def _fin_tail():
        tb = pl.multiple_of(c * HALF + HALF - 16, 8)
        pltpu.sync_copy(acc_sh.at[pl.ds(tb, 16)], buf_v.at[pl.ds(0, 16)])
        pltpu.sync_copy(dinv_hbm.at[pl.ds(tb, 16)], dinv_sv.at[pl.ds(0, 16)])
        pltpu.sync_copy(fin_hbm.at[pl.ds(tb, 16)], fin_sv.at[pl.ds(0, 16)])
        outb_v[pl.ds(0, 16)] = (dinv_sv[pl.ds(0, 16)] * buf_v[pl.ds(0, 16)]
                                + fin_sv[pl.ds(0, 16)])
        pltpu.sync_copy(outb_v.at[pl.ds(0, 16)], out_hbm.at[pl.ds(tb, 16)])


def _scalar_final(zs, fin, dinv, src, dst, zero_vec):
    return pl.kernel(
        _scalar_final_body,
        out_type=jax.ShapeDtypeStruct((N,), jnp.float32),
        mesh=_mesh(),
        compiler_params=_params(),
        scratch_types=[
            pltpu.VMEM((N,), jnp.float32),
            pltpu.VMEM((EPT2,), jnp.int32),
            pltpu.VMEM((EPT2,), jnp.int32),
            [pltpu.VMEM((SCH,), jnp.int32) for _ in range(2)],
            pltpu.VMEM((STAIL,), jnp.int32),
            [pltpu.VMEM((SCH,), jnp.float32) for _ in range(2)],
            pltpu.VMEM((STRIPE,), jnp.float32),
            pltpu.VMEM((FS,), jnp.float32),
            pltpu.VMEM((FS,), jnp.float32),
            pltpu.VMEM((FS,), jnp.float32),
            pltpu.VMEM_SHARED((N,), jnp.float32),
            [pltpu.SemaphoreType.DMA for _ in range(2)],
        ],
    )(zs, fin, dinv, src, dst, zero_vec)


RCH = 128
RNCH = EPT // RCH
RTAIL = EPT - RNCH * RCH
NBUF = 2


def _row_scatter_body(rows_hbm, src_hbm, dst_hbm, zero_hbm, out_hbm,
                      src_all, dstbs, dstbt, rows_bufs, acc_sh,
                      gsems, dsems, csems):
    c = lax.axis_index("c")
    s = lax.axis_index("s")
    ebase = pl.multiple_of((c * NS + s) * EPT, 8)
    pltpu.sync_copy(src_hbm.at[pl.ds(ebase, EPT)], src_all)
    rows0 = rows_bufs[0]
    off0 = pl.multiple_of(s * STRIPE, 8)
    pltpu.sync_copy(zero_hbm, rows0)
    for t in range(STRIPE // RCH):
        pltpu.sync_copy(rows0, acc_sh.at[pl.ds(off0 + t * RCH, RCH)])
    rem = STRIPE - (STRIPE // RCH) * RCH
    pltpu.sync_copy(rows0.at[pl.ds(0, rem)],
                    acc_sh.at[pl.ds(off0 + STRIPE - rem, rem)])

    @pl.when(s == NS - 1)
    def _zero_tail():
        pltpu.sync_copy(rows0.at[pl.ds(0, TAIL)], acc_sh.at[pl.ds(N - TAIL, TAIL)])

    plsc.subcore_barrier()

    def issue(b, off):
        pltpu.async_copy(dst_hbm.at[pl.ds(ebase + off, RCH)], dstbs[b],
                         dsems[b])
        pltpu.async_copy(rows_hbm.at[src_all.at[pl.ds(off, RCH)]],
                         rows_bufs[b], gsems[b])

    def process(b, off):
        pltpu.make_async_copy(dst_hbm.at[pl.ds(ebase + off, RCH)], dstbs[b],
                              dsems[b]).wait()
        pltpu.make_async_copy(rows_hbm.at[src_all.at[pl.ds(off, RCH)]],
                              rows_bufs[b], gsems[b]).wait()
        pltpu.async_copy(rows_bufs[b], acc_sh.at[dstbs[b]], csems[b],
                         add=True)

    def wait_scatter(b):
        pltpu.make_async_copy(rows_bufs[b], acc_sh.at[dstbs[b]],
                              csems[b]).wait()

    issue(0, pl.multiple_of(0, 8))

    def pair(g, carry):
        o0 = pl.multiple_of(2 * g * RCH, 8)
        o1 = pl.multiple_of((2 * g + 1) * RCH, 8)
        o2 = pl.multiple_of((2 * g + 2) * RCH, 8)
        issue(1, o1)
        process(0, o0)
        wait_scatter(0)
        issue(0, o2)
        process(1, o1)
        wait_scatter(1)
        return carry

    lax.fori_loop(0, RNCH // 2 - 1, pair, 0)
    o76 = pl.multiple_of((RNCH - 2) * RCH, 8)
    o77 = pl.multiple_of((RNCH - 1) * RCH, 8)
    issue(1, o77)
    process(0, o76)
    wait_scatter(0)
    process(1, o77)
    wait_scatter(1)
    ot = pl.multiple_of(RNCH * RCH, 8)
    pltpu.sync_copy(dst_hbm.at[pl.ds(ebase + ot, RTAIL)], dstbt)
    pltpu.async_copy(rows_hbm.at[src_all.at[pl.ds(ot, RTAIL)]],
                     rows0.at[pl.ds(0, RTAIL)], gsems[0]).wait()
    pltpu.sync_copy(rows0.at[pl.ds(0, RTAIL)], acc_sh.at[dstbt], add=True)

    plsc.subcore_barrier()
    obase = pl.multiple_of(c * N, 8)
    for t in range(STRIPE // RCH):
        pltpu.sync_copy(acc_sh.at[pl.ds(off0 + t * RCH, RCH)], rows0)
        pltpu.sync_copy(rows0, out_hbm.at[pl.ds(obase + off0 + t * RCH, RCH)])
    pltpu.sync_copy(acc_sh.at[pl.ds(off0 + STRIPE - rem, rem)],
                    rows0.at[pl.ds(0, rem)])
    pltpu.sync_copy(rows0.at[pl.ds(0, rem)],
                    out_hbm.at[pl.ds(obase + off0 + STRIPE - rem, rem)])

    @pl.when(s == NS - 1)
    def _out_tail():
        rows1 = rows_bufs[1]
        pltpu.sync_copy(acc_sh.at[pl.ds(N - TAIL, TAIL)], rows1.at[pl.ds(0, TAIL)])
        pltpu.sync_copy(rows1.at[pl.ds(0, TAIL)],
                        out_hbm.at[pl.ds(obase + N - TAIL, TAIL)])


def _row_scatter(rows, src, dst, zero_rows):
    return pl.kernel(
        _row_scatter_body,
        out_type=jax.ShapeDtypeStruct((NC * N, D_HID), jnp.float32),
        mesh=_mesh(),
        compiler_params=_params(),
        scratch_types=[
            pltpu.VMEM((EPT,), jnp.int32),
            [pltpu.VMEM((RCH,), jnp.int32) for _ in range(NBUF)],
            pltpu.VMEM((RTAIL,), jnp.int32),
            [pltpu.VMEM((RCH, D_HID), jnp.float32) for _ in range(NBUF)],
            pltpu.VMEM_SHARED((N, D_HID), jnp.float32),
            [pltpu.SemaphoreType.DMA for _ in range(NBUF)],
            [pltpu.SemaphoreType.DMA for _ in range(NBUF)],
            [pltpu.SemaphoreType.DMA for _ in range(NBUF)],
        ],
    )(rows, src, dst, zero_rows)


_RB = 400
_NG = N // _RB


def _scale_body(deg2_ref, x_ref, w1_ref, w2_ref, wlin_ref,
                xws_ref, dinv_ref, wz_ref):
    deg = deg2_ref[0] + deg2_ref[1] + 1.0
    dinv = lax.rsqrt(deg)
    dinv_ref[...] = dinv
    xw = jnp.dot(x_ref[...], w1_ref[...], preferred_element_type=jnp.float32)
    xws_ref[...] = dinv * xw
    wz_ref[...] = jnp.dot(w2_ref[...], wlin_ref[...],
                          preferred_element_type=jnp.float32)


def _scale_stage(deg_parts, x, W1, W2, Wlin):
    return pl.pallas_call(
        _scale_body,
        grid=(_NG,),
        in_specs=[
            pl.BlockSpec((NC, _RB, 1), lambda i: (0, i, 0)),
            pl.BlockSpec((_RB, D_IN), lambda i: (i, 0)),
            pl.BlockSpec((D_IN, D_HID), lambda i: (0, 0)),
            pl.BlockSpec((D_HID, D_EMB), lambda i: (0, 0)),
            pl.BlockSpec((D_EMB, 1), lambda i: (0, 0)),
        ],
        out_specs=[
            pl.BlockSpec((_RB, D_HID), lambda i: (i, 0)),
            pl.BlockSpec((_RB, 1), lambda i: (i, 0)),
            pl.BlockSpec((D_HID, 1), lambda i: (0, 0)),
        ],
        out_shape=[
            jax.ShapeDtypeStruct((N, D_HID), jnp.float32),
            jax.ShapeDtypeStruct((N, 1), jnp.float32),
            jax.ShapeDtypeStruct((D_HID, 1), jnp.float32),
        ],
    )(deg_parts, x, W1, W2, Wlin)


def _mid_body(agg_ref, xws_ref, dinv_ref, b1_ref, wz_ref, b2_ref, wlin_ref,
              blin_ref, zs_ref, fin_ref):
    dinv = dinv_ref[...]
    pre = dinv * (agg_ref[0] + agg_ref[1] + xws_ref[...]) + b1_ref[...]
    h = jnp.maximum(pre, 0.0)
    z = jnp.dot(h, wz_ref[...], preferred_element_type=jnp.float32)
    zs = dinv * z
    zs_ref[...] = zs
    cval = jnp.dot(b2_ref[...], wlin_ref[...],
                   preferred_element_type=jnp.float32) + blin_ref[...]
    fin_ref[...] = dinv * zs + cval


def _mid_stage(agg_parts, xws, dinv, b1, wz, b2, Wlin, blin):
    return pl.pallas_call(
        _mid_body,
        grid=(_NG,),
        in_specs=[
            pl.BlockSpec((NC, _RB, D_HID), lambda i: (0, i, 0)),
            pl.BlockSpec((_RB, D_HID), lambda i: (i, 0)),
            pl.BlockSpec((_RB, 1), lambda i: (i, 0)),
            pl.BlockSpec((1, D_HID), lambda i: (0, 0)),
            pl.BlockSpec((D_HID, 1), lambda i: (0, 0)),
            pl.BlockSpec((1, D_EMB), lambda i: (0, 0)),
            pl.BlockSpec((D_EMB, 1), lambda i: (0, 0)),
            pl.BlockSpec((1, 1), lambda i: (0, 0)),
        ],
        out_specs=[
            pl.BlockSpec((_RB, 1), lambda i: (i, 0)),
            pl.BlockSpec((_RB, 1), lambda i: (i, 0)),
        ],
        out_shape=[
            jax.ShapeDtypeStruct((N, 1), jnp.float32),
            jax.ShapeDtypeStruct((N, 1), jnp.float32),
        ],
    )(agg_parts, xws, dinv, b1, wz, b2, Wlin, blin)


def kernel(x, edge_index, W1, b1, W2, b2, Wlin, blin):
    src = edge_index[0]
    dst = edge_index[1]
    zero_vec = jnp.zeros((STRIPE,), jnp.float32)
    zero_rows = jnp.zeros((RCH, D_HID), jnp.float32)

    deg_parts = _deg_counts(dst, zero_vec)
    xws, dinv, wz = _scale_stage(deg_parts.reshape(NC, N, 1), x, W1, W2, Wlin)
    agg_parts = _row_scatter(xws, src, dst, zero_rows)
    zs, fin = _mid_stage(agg_parts.reshape(NC, N, D_HID), xws, dinv,
                         b1.reshape(1, D_HID), wz,
                         b2.reshape(1, D_EMB), Wlin, blin.reshape(1, 1))
    out = _scalar_final(zs.reshape(-1), fin.reshape(-1), dinv.reshape(-1),
                        src, dst, zero_vec)
    return out
